# Initial kernel scaffold; baseline (speedup 1.0000x reference)
import jax
import jax.numpy as jnp
from jax.experimental import pallas as pl


def kernel(xyz, dir0, w1, b1, dir1, fc1_w, fc1_b, fc2_w, fc2_b, fcd1_w, fcd1_b, fcd2_w, fcd2_b, fcg1_w, fcg1_b, fcg2_w, fcg2_b, wq, wk, wv):
    raise NotImplementedError("write your pallas kernel here")


# speedup vs baseline: 10.3338x; 10.3338x over previous
import functools

import jax
import jax.numpy as jnp
import numpy as np
from jax import lax
from jax.experimental import pallas as pl
from jax.experimental.pallas import tpu as pltpu
from jax.experimental.pallas import tpu_sc as plsc

_B = 4
_N = 2048
_TOP = 21
_NBR = 20
_KATT = 16
_D = 128
_PC = 128
_PCS = 16
_BLK = 256
_IDXW = 32

def _mm(a, b):
    return jnp.dot(a.astype(jnp.bfloat16), b.astype(jnp.bfloat16),
                   preferred_element_type=jnp.float32)


def _r(x):
    return x.astype(jnp.bfloat16).astype(jnp.float32)


_SC_NC, _SC_NS = 2, 16
_NW = _SC_NC * _SC_NS
_CH = 128


def _topk_body(xyzT_ref, vert_ref, idx_ref):
    b = pl.program_id(0)
    va = xyzT_ref[0]
    vb = vert_ref[0]
    va0, va1, va2 = va[0:1, :], va[1:2, :], va[2:3, :]
    vb0, vb1, vb2 = vb[:, 0:1], vb[:, 1:2], vb[:, 2:3]
    inner = (_r(vb0) * _r(va0) + _r(vb1) * _r(va1)
             + _r(vb2) * _r(va2))
    qa = va0 * va0 + va1 * va1 + va2 * va2
    qb = vb0 * vb0 + vb1 * vb1 + vb2 * vb2
    d = -2.0 * inner + qa + qb
    iota = lax.broadcasted_iota(jnp.int32, (_BLK, _N), 1)
    cols = []
    for _ in range(_TOP):
        m = jnp.min(d, axis=1, keepdims=True)
        cand = jnp.where(d == m, iota, _N)
        i = jnp.min(cand, axis=1, keepdims=True)
        cols.append(i)
        d = jnp.where(iota == i, jnp.inf, d)
    idx = jnp.concatenate(
        cols + [jnp.zeros((_BLK, _IDXW - _TOP), jnp.int32)], axis=1)
    idx_ref[0] = idx + b * _N


def _topk(xyz, vertices):
    return pl.pallas_call(
        _topk_body,
        grid=(_B, _N // _BLK),
        in_specs=[
            pl.BlockSpec((1, 3, _N), lambda b, n: (b, 0, 0)),
            pl.BlockSpec((1, _BLK, 3), lambda b, n: (b, n, 0)),
        ],
        out_specs=pl.BlockSpec((1, _BLK, _IDXW), lambda b, n: (b, n, 0)),
        out_shape=jax.ShapeDtypeStruct((_B, _N, _IDXW), jnp.int32),
    )(xyz, vertices)


def _sc_gather(tables, idx_flat):
    nt = len(tables)
    rows = idx_flat.shape[0]
    per_w = rows // _NW
    n_ch = per_w // _CH
    mesh = plsc.VectorSubcoreMesh(core_axis_name="c", subcore_axis_name="s")
    out_type = [jax.ShapeDtypeStruct((rows, t.shape[1]), t.dtype)
                for t in tables]
    scratch = ([pltpu.VMEM((_CH,), jnp.int32)]
               + [pltpu.VMEM((_CH, t.shape[1]), jnp.float32) for t in tables]
               + [pltpu.SemaphoreType.DMA])

    @functools.partial(pl.kernel, mesh=mesh, out_type=out_type,
                       scratch_types=scratch)
    def gath(idx_hbm, *rest):
        tabs = rest[:nt]
        outs = rest[nt:2 * nt]
        idx_v = rest[2 * nt]
        bufs = rest[2 * nt + 1:2 * nt + 1 + nt]
        sem = rest[-1]
        wid = lax.axis_index("s") * _SC_NC + lax.axis_index("c")
        base = wid * per_w

        def body(i, carry):
            off = base + i * _CH
            pltpu.sync_copy(idx_hbm.at[pl.ds(off, _CH)], idx_v)
            for t in range(nt):
                pltpu.async_copy(tabs[t].at[idx_v], bufs[t], sem).wait()
                pltpu.sync_copy(bufs[t], outs[t].at[pl.ds(off, _CH)])
            return carry

        lax.fori_loop(0, n_ch, body, 0)

    res = gath(idx_flat, *tables)
    return res if isinstance(res, (list, tuple)) else [res]


def _conv1_body(nbr_ref, vert_ref, dir0_ref, w1_ref, b1_ref,
                cen_ref, supp_ref, ndn_ref):
    vert = vert_ref[0]
    d0 = dir0_ref[...]
    n0 = jnp.sqrt(jnp.sum(d0 * d0, axis=0, keepdims=True))
    d0p = jnp.concatenate(
        [d0 / jnp.maximum(n0, 1e-12),
         jnp.zeros((_PC - 3, _D), jnp.float32)], axis=0)
    acc = None
    for j in range(1, _TOP):
        diff = nbr_ref[0, :, j, :] - vert
        nrm = jnp.sqrt(jnp.sum(diff * diff, axis=1, keepdims=True))
        ndn = diff / jnp.maximum(nrm, 1e-12)
        ndn_ref[0, :, j - 1, :] = ndn[:, :_PCS]
        th = jnp.maximum(
            _mm(ndn, d0p), 0.0)
        acc = th if acc is None else jnp.maximum(acc, th)
    fm = jnp.maximum(acc, 0.0)
    fo = _mm(fm, w1_ref[...]) + b1_ref[...]
    cen_ref[0] = fo[:, :_D]
    supp_ref[0] = fo[:, _D:]


def _conv1(nbr, vert_pad, dir0, w1, b1r):
    blkmap = lambda b, n: (b, n, 0)
    return pl.pallas_call(
        _conv1_body,
        grid=(_B, _N // _BLK),
        in_specs=[
            pl.BlockSpec((1, _BLK, _TOP, _PC), lambda b, n: (b, n, 0, 0)),
            pl.BlockSpec((1, _BLK, _PC), blkmap),
            pl.BlockSpec((3, _D), lambda b, n: (0, 0)),
            pl.BlockSpec((_D, 2 * _D), lambda b, n: (0, 0)),
            pl.BlockSpec((1, 2 * _D), lambda b, n: (0, 0)),
        ],
        out_specs=[
            pl.BlockSpec((1, _BLK, _D), blkmap),
            pl.BlockSpec((1, _BLK, _D), blkmap),
            pl.BlockSpec((1, _BLK, _NBR, _PCS), lambda b, n: (b, n, 0, 0)),
        ],
        out_shape=[
            jax.ShapeDtypeStruct((_B, _N, _D), jnp.float32),
            jax.ShapeDtypeStruct((_B, _N, _D), jnp.float32),
            jax.ShapeDtypeStruct((_B, _N, _NBR, _PCS), jnp.float32),
        ],
    )(nbr, vert_pad, dir0, w1, b1r)


def _conv2_body(ndn_ref, sn_ref, cen_ref, dir1_ref, fc1w_ref, fc1b_ref,
                wq_ref, wk_ref, wv_ref, q_ref, k_ref, v_ref, fm2_ref):
    d1 = dir1_ref[...]
    n1 = jnp.sqrt(jnp.sum(d1 * d1, axis=0, keepdims=True))
    d1p = jnp.concatenate(
        [d1 / jnp.maximum(n1, 1e-12),
         jnp.zeros((_PCS - 3, _D), jnp.float32)], axis=0)
    acc = None
    for j in range(_NBR):
        th1 = jnp.maximum(_mm(ndn_ref[0, :, j, :], d1p), 0.0)
        a = th1 * sn_ref[0, :, j, :]
        acc = a if acc is None else jnp.maximum(acc, a)
    fm2 = jnp.maximum(cen_ref[0] + acc, 0.0)
    x = _mm(fm2, fc1w_ref[...]) + fc1b_ref[...]
    q_ref[0] = _mm(x, wq_ref[...])
    k_ref[0] = _mm(x, wk_ref[...])
    v_ref[0] = _mm(x, wv_ref[...])
    fm2_ref[0] = fm2


def _conv2(ndn, sn, cen, dir1, fc1_w, fc1br, wq, wk, wv):
    blkmap = lambda b, n: (b, n, 0)
    wmap = lambda b, n: (0, 0)
    od = jax.ShapeDtypeStruct((_B, _N, _D), jnp.float32)
    return pl.pallas_call(
        _conv2_body,
        grid=(_B, _N // _BLK),
        in_specs=[
            pl.BlockSpec((1, _BLK, _NBR, _PCS), lambda b, n: (b, n, 0, 0)),
            pl.BlockSpec((1, _BLK, _NBR, _D), lambda b, n: (b, n, 0, 0)),
            pl.BlockSpec((1, _BLK, _D), blkmap),
            pl.BlockSpec((3, _D), wmap),
            pl.BlockSpec((_D, _D), wmap),
            pl.BlockSpec((1, _D), wmap),
            pl.BlockSpec((_D, _D), wmap),
            pl.BlockSpec((_D, _D), wmap),
            pl.BlockSpec((_D, _D), wmap),
        ],
        out_specs=[pl.BlockSpec((1, _BLK, _D), blkmap)] * 4,
        out_shape=[od, od, od, od],
    )(ndn, sn, cen, dir1, fc1_w, fc1br, wq, wk, wv)


def _attn_body(nbr_ref, vert_ref, q_ref, kk_ref, vv_ref, fm2_ref,
               fcd1_ref, fcd1b_ref, fcd2_ref, fcd2b_ref,
               fcg1_ref, fcg1b_ref, fcg2_ref, fcg2b_ref,
               fc2w_ref, fc2b_ref, out_ref):
    vert = vert_ref[0]
    fd1p = jnp.concatenate(
        [fcd1_ref[...], jnp.zeros((_PC - 3, _D), jnp.float32)], axis=0)
    q = q_ref[0]
    scale = 1.0 / np.sqrt(float(_D))
    zs, ps = [], []
    m = None
    for j in range(_KATT):
        rel = vert - nbr_ref[0, :, j, :]
        h = jnp.maximum(
            _mm(rel, fd1p)
            + fcd1b_ref[...], 0.0)
        pos = _mm(h, fcd2_ref[...]) + fcd2b_ref[...]
        t = q - kk_ref[0, :, j, :] + pos
        g = jnp.maximum(
            _mm(t, fcg1_ref[...])
            + fcg1b_ref[...], 0.0)
        z = (_mm(g, fcg2_ref[...])
             + fcg2b_ref[...]) * scale
        zs.append(z)
        ps.append(pos)
        m = z if m is None else jnp.maximum(m, z)
    es, s = [], None
    for j in range(_KATT):
        e = jnp.exp(zs[j] - m)
        es.append(e)
        s = e if s is None else s + e
    res = None
    for j in range(_KATT):
        c = (es[j] / s) * (vv_ref[0, :, j, :] + ps[j])
        res = c if res is None else res + c
    out_ref[0] = (_mm(res, fc2w_ref[...])
                  + fc2b_ref[...] + fm2_ref[0])


def _attn(nbr, vert_pad, q, kk, vv, fm2,
          fcd1, fcd1br, fcd2, fcd2br, fcg1, fcg1br, fcg2, fcg2br,
          fc2_w, fc2br):
    blkmap = lambda b, n: (b, n, 0)
    wmap = lambda b, n: (0, 0)
    return pl.pallas_call(
        _attn_body,
        grid=(_B, _N // _BLK),
        in_specs=[
            pl.BlockSpec((1, _BLK, _TOP, _PC), lambda b, n: (b, n, 0, 0)),
            pl.BlockSpec((1, _BLK, _PC), blkmap),
            pl.BlockSpec((1, _BLK, _D), blkmap),
            pl.BlockSpec((1, _BLK, _KATT, _D), lambda b, n: (b, n, 0, 0)),
            pl.BlockSpec((1, _BLK, _KATT, _D), lambda b, n: (b, n, 0, 0)),
            pl.BlockSpec((1, _BLK, _D), blkmap),
            pl.BlockSpec((3, _D), wmap),
            pl.BlockSpec((1, _D), wmap),
            pl.BlockSpec((_D, _D), wmap),
            pl.BlockSpec((1, _D), wmap),
            pl.BlockSpec((_D, _D), wmap),
            pl.BlockSpec((1, _D), wmap),
            pl.BlockSpec((_D, _D), wmap),
            pl.BlockSpec((1, _D), wmap),
            pl.BlockSpec((_D, _D), wmap),
            pl.BlockSpec((1, _D), wmap),
        ],
        out_specs=pl.BlockSpec((1, _BLK, _D), blkmap),
        out_shape=jax.ShapeDtypeStruct((_B, _N, _D), jnp.float32),
    )(nbr, vert_pad, q, kk, vv, fm2,
      fcd1, fcd1br, fcd2, fcd2br, fcg1, fcg1br, fcg2, fcg2br,
      fc2_w, fc2br)


def kernel(xyz, dir0, w1, b1, dir1, fc1_w, fc1_b, fc2_w, fc2_b,
           fcd1_w, fcd1_b, fcd2_w, fcd2_b, fcg1_w, fcg1_b, fcg2_w, fcg2_b,
           wq, wk, wv):
    vertices = jnp.transpose(xyz, (0, 2, 1))
    vert_pad = jnp.pad(vertices, ((0, 0), (0, 0), (0, _PC - 3)))

    idx = _topk(xyz, vertices)

    idx21 = idx[:, :, :_TOP].reshape(-1)
    (nbr_flat,) = _sc_gather([vert_pad.reshape(_B * _N, _PC)], idx21)
    nbr = nbr_flat.reshape(_B, _N, _TOP, _PC)

    cen, supp, ndn = _conv1(nbr, vert_pad, dir0, w1, b1.reshape(1, -1))

    idx20 = idx[:, :, 1:_TOP].reshape(-1)
    (sn_flat,) = _sc_gather([supp.reshape(_B * _N, _D)], idx20)
    sn = sn_flat.reshape(_B, _N, _NBR, _D)

    q, kx, vx, fm2 = _conv2(ndn, sn, cen, dir1, fc1_w,
                            fc1_b.reshape(1, -1), wq, wk, wv)

    idx16 = idx[:, :, :_KATT].reshape(-1)
    kkf, vvf = _sc_gather(
        [kx.reshape(_B * _N, _D), vx.reshape(_B * _N, _D)], idx16)
    kk = kkf.reshape(_B, _N, _KATT, _D)
    vv = vvf.reshape(_B, _N, _KATT, _D)

    out = _attn(nbr, vert_pad, q, kk, vv, fm2,
                fcd1_w, fcd1_b.reshape(1, -1), fcd2_w, fcd2_b.reshape(1, -1),
                fcg1_w, fcg1_b.reshape(1, -1), fcg2_w, fcg2_b.reshape(1, -1),
                fc2_w, fc2_b.reshape(1, -1))
    return jnp.transpose(out, (0, 2, 1))


# speedup vs baseline: 16.2863x; 1.5760x over previous
import functools

import jax
import jax.numpy as jnp
import numpy as np
from jax import lax
from jax.experimental import pallas as pl
from jax.experimental.pallas import tpu as pltpu
from jax.experimental.pallas import tpu_sc as plsc

_B = 4
_N = 2048
_TOP = 21
_NBR = 20
_KATT = 16
_D = 128
_PC = 128
_PCS = 16
_BLK = 256
_IDXW = 32

def _mm(a, b):
    return jnp.dot(a.astype(jnp.bfloat16), b.astype(jnp.bfloat16),
                   preferred_element_type=jnp.float32)


def _r(x):
    return x.astype(jnp.bfloat16).astype(jnp.float32)


_SC_NC, _SC_NS = 2, 16
_NW = _SC_NC * _SC_NS
_CH = 128


def _topk_body(xyzT_ref, vert_ref, idx_ref):
    b = pl.program_id(0)
    va = xyzT_ref[0]
    vb = vert_ref[0]
    va0, va1, va2 = va[0:1, :], va[1:2, :], va[2:3, :]
    vb0, vb1, vb2 = vb[:, 0:1], vb[:, 1:2], vb[:, 2:3]
    inner = (_r(vb0) * _r(va0) + _r(vb1) * _r(va1)
             + _r(vb2) * _r(va2))
    qa = va0 * va0 + va1 * va1 + va2 * va2
    qb = vb0 * vb0 + vb1 * vb1 + vb2 * vb2
    d = -2.0 * inner + qa + qb
    iota = lax.broadcasted_iota(
        jnp.int32, (_BLK, _N), 1).astype(jnp.float32)
    fn = jnp.float32(_N)
    cols = []
    for _ in range(_TOP):
        m = jnp.min(d, axis=1, keepdims=True)
        cand = jnp.where(d == m, iota, fn)
        i = jnp.min(cand, axis=1, keepdims=True)
        cols.append(i)
        d = jnp.where(iota == i, jnp.inf, d)
    idx = jnp.concatenate(cols, axis=1).astype(jnp.int32)
    idx = jnp.concatenate(
        [idx, jnp.zeros((_BLK, _IDXW - _TOP), jnp.int32)], axis=1)
    idx_ref[0] = idx + b * _N


def _topk(xyz, vertices):
    return pl.pallas_call(
        _topk_body,
        grid=(_B, _N // _BLK),
        in_specs=[
            pl.BlockSpec((1, 3, _N), lambda b, n: (b, 0, 0)),
            pl.BlockSpec((1, _BLK, 3), lambda b, n: (b, n, 0)),
        ],
        out_specs=pl.BlockSpec((1, _BLK, _IDXW), lambda b, n: (b, n, 0)),
        out_shape=jax.ShapeDtypeStruct((_B, _N, _IDXW), jnp.int32),
    )(xyz, vertices)


def _sc_gather(tables, idx_flat):
    nt = len(tables)
    rows = idx_flat.shape[0]
    per_w = rows // _NW
    n_ch = per_w // _CH
    mesh = plsc.VectorSubcoreMesh(core_axis_name="c", subcore_axis_name="s")
    out_type = [jax.ShapeDtypeStruct((rows, t.shape[1]), t.dtype)
                for t in tables]
    scratch = ([pltpu.VMEM((_CH,), jnp.int32)]
               + [pltpu.VMEM((_CH, t.shape[1]), jnp.float32) for t in tables]
               + [pltpu.SemaphoreType.DMA])

    @functools.partial(pl.kernel, mesh=mesh, out_type=out_type,
                       scratch_types=scratch)
    def gath(idx_hbm, *rest):
        tabs = rest[:nt]
        outs = rest[nt:2 * nt]
        idx_v = rest[2 * nt]
        bufs = rest[2 * nt + 1:2 * nt + 1 + nt]
        sem = rest[-1]
        wid = lax.axis_index("s") * _SC_NC + lax.axis_index("c")
        base = wid * per_w

        def body(i, carry):
            off = base + i * _CH
            pltpu.sync_copy(idx_hbm.at[pl.ds(off, _CH)], idx_v)
            for t in range(nt):
                pltpu.async_copy(tabs[t].at[idx_v], bufs[t], sem).wait()
                pltpu.sync_copy(bufs[t], outs[t].at[pl.ds(off, _CH)])
            return carry

        lax.fori_loop(0, n_ch, body, 0)

    res = gath(idx_flat, *tables)
    return res if isinstance(res, (list, tuple)) else [res]


def _conv1_body(nbr_ref, vert_ref, dir0_ref, w1_ref, b1_ref,
                cen_ref, supp_ref, ndn_ref):
    vert = vert_ref[0]
    d0 = dir0_ref[...]
    n0 = jnp.sqrt(jnp.sum(d0 * d0, axis=0, keepdims=True))
    d0p = jnp.concatenate(
        [d0 / jnp.maximum(n0, 1e-12),
         jnp.zeros((_PC - 3, _D), jnp.float32)], axis=0)
    acc = None
    for j in range(1, _TOP):
        diff = nbr_ref[j, 0] - vert
        nrm = jnp.sqrt(jnp.sum(diff * diff, axis=1, keepdims=True))
        ndn = diff / jnp.maximum(nrm, 1e-12)
        ndn_ref[j - 1, 0] = ndn[:, :_PCS]
        th = jnp.maximum(
            _mm(ndn, d0p), 0.0)
        acc = th if acc is None else jnp.maximum(acc, th)
    fm = jnp.maximum(acc, 0.0)
    fo = _mm(fm, w1_ref[...]) + b1_ref[...]
    cen_ref[0] = fo[:, :_D]
    supp_ref[0] = fo[:, _D:]


def _conv1(nbr, vert_pad, dir0, w1, b1r):
    blkmap = lambda b, n: (b, n, 0)
    return pl.pallas_call(
        _conv1_body,
        grid=(_B, _N // _BLK),
        in_specs=[
            pl.BlockSpec((_TOP, 1, _BLK, _PC), lambda b, n: (0, b, n, 0)),
            pl.BlockSpec((1, _BLK, _PC), blkmap),
            pl.BlockSpec((3, _D), lambda b, n: (0, 0)),
            pl.BlockSpec((_D, 2 * _D), lambda b, n: (0, 0)),
            pl.BlockSpec((1, 2 * _D), lambda b, n: (0, 0)),
        ],
        out_specs=[
            pl.BlockSpec((1, _BLK, _D), blkmap),
            pl.BlockSpec((1, _BLK, _D), blkmap),
            pl.BlockSpec((_NBR, 1, _BLK, _PCS), lambda b, n: (0, b, n, 0)),
        ],
        out_shape=[
            jax.ShapeDtypeStruct((_B, _N, _D), jnp.float32),
            jax.ShapeDtypeStruct((_B, _N, _D), jnp.float32),
            jax.ShapeDtypeStruct((_NBR, _B, _N, _PCS), jnp.float32),
        ],
    )(nbr, vert_pad, dir0, w1, b1r)


def _conv2_body(ndn_ref, sn_ref, cen_ref, dir1_ref, fc1w_ref, fc1b_ref,
                wq_ref, wk_ref, wv_ref, q_ref, k_ref, v_ref, fm2_ref):
    d1 = dir1_ref[...]
    n1 = jnp.sqrt(jnp.sum(d1 * d1, axis=0, keepdims=True))
    d1p = jnp.concatenate(
        [d1 / jnp.maximum(n1, 1e-12),
         jnp.zeros((_PCS - 3, _D), jnp.float32)], axis=0)
    acc = None
    for j in range(_NBR):
        th1 = jnp.maximum(_mm(ndn_ref[j, 0], d1p), 0.0)
        a = th1 * sn_ref[j, 0]
        acc = a if acc is None else jnp.maximum(acc, a)
    fm2 = jnp.maximum(cen_ref[0] + acc, 0.0)
    x = _mm(fm2, fc1w_ref[...]) + fc1b_ref[...]
    q_ref[0] = _mm(x, wq_ref[...])
    k_ref[0] = _mm(x, wk_ref[...])
    v_ref[0] = _mm(x, wv_ref[...])
    fm2_ref[0] = fm2


def _conv2(ndn, sn, cen, dir1, fc1_w, fc1br, wq, wk, wv):
    blkmap = lambda b, n: (b, n, 0)
    wmap = lambda b, n: (0, 0)
    od = jax.ShapeDtypeStruct((_B, _N, _D), jnp.float32)
    return pl.pallas_call(
        _conv2_body,
        grid=(_B, _N // _BLK),
        in_specs=[
            pl.BlockSpec((_NBR, 1, _BLK, _PCS), lambda b, n: (0, b, n, 0)),
            pl.BlockSpec((_NBR, 1, _BLK, _D), lambda b, n: (0, b, n, 0)),
            pl.BlockSpec((1, _BLK, _D), blkmap),
            pl.BlockSpec((3, _D), wmap),
            pl.BlockSpec((_D, _D), wmap),
            pl.BlockSpec((1, _D), wmap),
            pl.BlockSpec((_D, _D), wmap),
            pl.BlockSpec((_D, _D), wmap),
            pl.BlockSpec((_D, _D), wmap),
        ],
        out_specs=[pl.BlockSpec((1, _BLK, _D), blkmap)] * 4,
        out_shape=[od, od, od, od],
    )(ndn, sn, cen, dir1, fc1_w, fc1br, wq, wk, wv)


def _attn_body(nbr_ref, vert_ref, q_ref, kk_ref, vv_ref, fm2_ref,
               fcd1_ref, fcd1b_ref, fcd2_ref, fcd2b_ref,
               fcg1_ref, fcg1b_ref, fcg2_ref, fcg2b_ref,
               fc2w_ref, fc2b_ref, out_ref):
    vert = vert_ref[0]
    fd1p = jnp.concatenate(
        [fcd1_ref[...], jnp.zeros((_PC - 3, _D), jnp.float32)], axis=0)
    q = q_ref[0]
    scale = 1.0 / np.sqrt(float(_D))
    zs, ps = [], []
    m = None
    for j in range(_KATT):
        rel = vert - nbr_ref[j, 0]
        h = jnp.maximum(
            _mm(rel, fd1p)
            + fcd1b_ref[...], 0.0)
        pos = _mm(h, fcd2_ref[...]) + fcd2b_ref[...]
        t = q - kk_ref[j, 0] + pos
        g = jnp.maximum(
            _mm(t, fcg1_ref[...])
            + fcg1b_ref[...], 0.0)
        z = (_mm(g, fcg2_ref[...])
             + fcg2b_ref[...]) * scale
        zs.append(z)
        ps.append(pos)
        m = z if m is None else jnp.maximum(m, z)
    es, s = [], None
    for j in range(_KATT):
        e = jnp.exp(zs[j] - m)
        es.append(e)
        s = e if s is None else s + e
    res = None
    for j in range(_KATT):
        c = (es[j] / s) * (vv_ref[j, 0] + ps[j])
        res = c if res is None else res + c
    out_ref[0] = (_mm(res, fc2w_ref[...])
                  + fc2b_ref[...] + fm2_ref[0])


def _attn(nbr, vert_pad, q, kk, vv, fm2,
          fcd1, fcd1br, fcd2, fcd2br, fcg1, fcg1br, fcg2, fcg2br,
          fc2_w, fc2br):
    blkmap = lambda b, n: (b, n, 0)
    wmap = lambda b, n: (0, 0)
    return pl.pallas_call(
        _attn_body,
        grid=(_B, _N // _BLK),
        in_specs=[
            pl.BlockSpec((_TOP, 1, _BLK, _PC), lambda b, n: (0, b, n, 0)),
            pl.BlockSpec((1, _BLK, _PC), blkmap),
            pl.BlockSpec((1, _BLK, _D), blkmap),
            pl.BlockSpec((_KATT, 1, _BLK, _D), lambda b, n: (0, b, n, 0)),
            pl.BlockSpec((_KATT, 1, _BLK, _D), lambda b, n: (0, b, n, 0)),
            pl.BlockSpec((1, _BLK, _D), blkmap),
            pl.BlockSpec((3, _D), wmap),
            pl.BlockSpec((1, _D), wmap),
            pl.BlockSpec((_D, _D), wmap),
            pl.BlockSpec((1, _D), wmap),
            pl.BlockSpec((_D, _D), wmap),
            pl.BlockSpec((1, _D), wmap),
            pl.BlockSpec((_D, _D), wmap),
            pl.BlockSpec((1, _D), wmap),
            pl.BlockSpec((_D, _D), wmap),
            pl.BlockSpec((1, _D), wmap),
        ],
        out_specs=pl.BlockSpec((1, _BLK, _D), blkmap),
        out_shape=jax.ShapeDtypeStruct((_B, _N, _D), jnp.float32),
    )(nbr, vert_pad, q, kk, vv, fm2,
      fcd1, fcd1br, fcd2, fcd2br, fcg1, fcg1br, fcg2, fcg2br,
      fc2_w, fc2br)


def kernel(xyz, dir0, w1, b1, dir1, fc1_w, fc1_b, fc2_w, fc2_b,
           fcd1_w, fcd1_b, fcd2_w, fcd2_b, fcg1_w, fcg1_b, fcg2_w, fcg2_b,
           wq, wk, wv):
    vertices = jnp.transpose(xyz, (0, 2, 1))
    vert_pad = jnp.pad(vertices, ((0, 0), (0, 0), (0, _PC - 3)))

    idx = _topk(xyz, vertices)

    idx21 = jnp.transpose(idx[:, :, :_TOP], (2, 0, 1)).reshape(-1)
    (nbr_flat,) = _sc_gather([vert_pad.reshape(_B * _N, _PC)], idx21)
    nbr = nbr_flat.reshape(_TOP, _B, _N, _PC)

    cen, supp, ndn = _conv1(nbr, vert_pad, dir0,
                            w1.astype(jnp.bfloat16), b1.reshape(1, -1))

    idx20 = jnp.transpose(idx[:, :, 1:_TOP], (2, 0, 1)).reshape(-1)
    (sn_flat,) = _sc_gather([supp.reshape(_B * _N, _D)], idx20)
    sn = sn_flat.reshape(_NBR, _B, _N, _D)

    q, kx, vx, fm2 = _conv2(ndn, sn, cen, dir1,
                            fc1_w.astype(jnp.bfloat16),
                            fc1_b.reshape(1, -1),
                            wq.astype(jnp.bfloat16),
                            wk.astype(jnp.bfloat16),
                            wv.astype(jnp.bfloat16))

    idx16 = jnp.transpose(idx[:, :, :_KATT], (2, 0, 1)).reshape(-1)
    kkf, vvf = _sc_gather(
        [kx.reshape(_B * _N, _D), vx.reshape(_B * _N, _D)], idx16)
    kk = kkf.reshape(_KATT, _B, _N, _D)
    vv = vvf.reshape(_KATT, _B, _N, _D)

    out = _attn(nbr, vert_pad, q, kk, vv, fm2,
                fcd1_w, fcd1_b.reshape(1, -1),
                fcd2_w.astype(jnp.bfloat16), fcd2_b.reshape(1, -1),
                fcg1_w.astype(jnp.bfloat16), fcg1_b.reshape(1, -1),
                fcg2_w.astype(jnp.bfloat16), fcg2_b.reshape(1, -1),
                fc2_w.astype(jnp.bfloat16), fc2_b.reshape(1, -1))
    return jnp.transpose(out, (0, 2, 1))


# speedup vs baseline: 18.8148x; 1.1553x over previous
import functools

import jax
import jax.numpy as jnp
import numpy as np
from jax import lax
from jax.experimental import pallas as pl
from jax.experimental.pallas import tpu as pltpu
from jax.experimental.pallas import tpu_sc as plsc

_B = 4
_N = 2048
_TOP = 21
_NBR = 20
_KATT = 16
_D = 128
_PC = 128
_PCS = 16
_BLK = 256
_IDXW = 32

def _mm(a, b):
    return jnp.dot(a.astype(jnp.bfloat16), b.astype(jnp.bfloat16),
                   preferred_element_type=jnp.float32)


def _r(x):
    return x.astype(jnp.bfloat16).astype(jnp.float32)


_SC_NC, _SC_NS = 2, 16
_NW = _SC_NC * _SC_NS
_CH = 128


def _topk_body(xyzT_ref, vert_ref, idx_ref):
    b = pl.program_id(0)
    va = xyzT_ref[0]
    vb = vert_ref[0]
    va0, va1, va2 = va[0:1, :], va[1:2, :], va[2:3, :]
    vb0, vb1, vb2 = vb[:, 0:1], vb[:, 1:2], vb[:, 2:3]
    inner = (_r(vb0) * _r(va0) + _r(vb1) * _r(va1)
             + _r(vb2) * _r(va2))
    qa = va0 * va0 + va1 * va1 + va2 * va2
    qb = vb0 * vb0 + vb1 * vb1 + vb2 * vb2
    d = -2.0 * inner + qa + qb
    iota = lax.broadcasted_iota(
        jnp.int32, (_BLK, _N), 1).astype(jnp.float32)
    fn = jnp.float32(_N)
    cols = []
    for _ in range(_TOP):
        m = jnp.min(d, axis=1, keepdims=True)
        cand = jnp.where(d == m, iota, fn)
        i = jnp.min(cand, axis=1, keepdims=True)
        cols.append(i)
        d = jnp.where(iota == i, jnp.inf, d)
    idx = jnp.concatenate(cols, axis=1).astype(jnp.int32)
    idx = jnp.concatenate(
        [idx, jnp.zeros((_BLK, _IDXW - _TOP), jnp.int32)], axis=1)
    idx_ref[0] = idx + b * _N


def _topk(xyz, vertices):
    return pl.pallas_call(
        _topk_body,
        grid=(_B, _N // _BLK),
        in_specs=[
            pl.BlockSpec((1, 3, _N), lambda b, n: (b, 0, 0)),
            pl.BlockSpec((1, _BLK, 3), lambda b, n: (b, n, 0)),
        ],
        out_specs=pl.BlockSpec((1, _BLK, _IDXW), lambda b, n: (b, n, 0)),
        out_shape=jax.ShapeDtypeStruct((_B, _N, _IDXW), jnp.int32),
    )(xyz, vertices)


def _sc_gather(tables, idx_flat):
    nt = len(tables)
    rows = idx_flat.shape[0]
    per_w = rows // _NW
    n_ch = per_w // _CH
    mesh = plsc.VectorSubcoreMesh(core_axis_name="c", subcore_axis_name="s")
    out_type = [jax.ShapeDtypeStruct((rows, t.shape[1]), t.dtype)
                for t in tables]
    assert n_ch % 2 == 0
    scratch = ([pltpu.VMEM((per_w,), jnp.int32)]
               + [pltpu.VMEM((_CH, t.shape[1]), jnp.float32)
                  for t in tables for _ in range(2)]
               + [pltpu.SemaphoreType.DMA] * 4)

    @functools.partial(pl.kernel, mesh=mesh, out_type=out_type,
                       scratch_types=scratch)
    def gath(idx_hbm, *rest):
        tabs = rest[:nt]
        outs = rest[nt:2 * nt]
        idx_v = rest[2 * nt]
        bufs = rest[2 * nt + 1:2 * nt + 1 + 2 * nt]
        gsem = rest[-4:-2]
        ssem = rest[-2:]
        wid = lax.axis_index("s") * _SC_NC + lax.axis_index("c")
        base = wid * per_w
        pltpu.sync_copy(idx_hbm.at[pl.ds(base, per_w)], idx_v)

        def body(p, carry):
            gh, sh = [], []
            for s in range(2):
                c = 2 * p + s
                ix = idx_v.at[pl.ds(c * _CH, _CH)]
                gh.append([pltpu.async_copy(tabs[t].at[ix],
                                            bufs[2 * t + s], gsem[s])
                           for t in range(nt)])
            for s in range(2):
                c = 2 * p + s
                off = base + c * _CH
                for t in range(nt):
                    gh[s][t].wait()
                    sh.append(pltpu.async_copy(
                        bufs[2 * t + s], outs[t].at[pl.ds(off, _CH)],
                        ssem[s]))
            for h in sh:
                h.wait()
            return carry

        lax.fori_loop(0, n_ch // 2, body, 0)

    res = gath(idx_flat, *tables)
    return res if isinstance(res, (list, tuple)) else [res]


def _conv1_body(nbr_ref, vert_ref, dir0_ref, w1_ref, b1_ref,
                cen_ref, supp_ref, ndn_ref):
    vert = vert_ref[0]
    d0 = dir0_ref[...]
    n0 = jnp.sqrt(jnp.sum(d0 * d0, axis=0, keepdims=True))
    d0p = jnp.concatenate(
        [d0 / jnp.maximum(n0, 1e-12),
         jnp.zeros((_PC - 3, _D), jnp.float32)], axis=0)
    acc = None
    for j in range(1, _TOP):
        diff = nbr_ref[j, 0] - vert
        nrm = jnp.sqrt(jnp.sum(diff * diff, axis=1, keepdims=True))
        ndn = diff / jnp.maximum(nrm, 1e-12)
        ndn_ref[j - 1, 0] = ndn[:, :_PCS]
        th = jnp.maximum(
            _mm(ndn, d0p), 0.0)
        acc = th if acc is None else jnp.maximum(acc, th)
    fm = jnp.maximum(acc, 0.0)
    fo = _mm(fm, w1_ref[...]) + b1_ref[...]
    cen_ref[0] = fo[:, :_D]
    supp_ref[0] = fo[:, _D:]


def _conv1(nbr, vert_pad, dir0, w1, b1r):
    blkmap = lambda b, n: (b, n, 0)
    return pl.pallas_call(
        _conv1_body,
        grid=(_B, _N // _BLK),
        in_specs=[
            pl.BlockSpec((_TOP, 1, _BLK, _PC), lambda b, n: (0, b, n, 0)),
            pl.BlockSpec((1, _BLK, _PC), blkmap),
            pl.BlockSpec((3, _D), lambda b, n: (0, 0)),
            pl.BlockSpec((_D, 2 * _D), lambda b, n: (0, 0)),
            pl.BlockSpec((1, 2 * _D), lambda b, n: (0, 0)),
        ],
        out_specs=[
            pl.BlockSpec((1, _BLK, _D), blkmap),
            pl.BlockSpec((1, _BLK, _D), blkmap),
            pl.BlockSpec((_NBR, 1, _BLK, _PCS), lambda b, n: (0, b, n, 0)),
        ],
        out_shape=[
            jax.ShapeDtypeStruct((_B, _N, _D), jnp.float32),
            jax.ShapeDtypeStruct((_B, _N, _D), jnp.float32),
            jax.ShapeDtypeStruct((_NBR, _B, _N, _PCS), jnp.float32),
        ],
    )(nbr, vert_pad, dir0, w1, b1r)


def _conv2_body(ndn_ref, sn_ref, cen_ref, dir1_ref, fc1w_ref, fc1b_ref,
                wq_ref, wk_ref, wv_ref, q_ref, k_ref, v_ref, fm2_ref):
    d1 = dir1_ref[...]
    n1 = jnp.sqrt(jnp.sum(d1 * d1, axis=0, keepdims=True))
    d1p = jnp.concatenate(
        [d1 / jnp.maximum(n1, 1e-12),
         jnp.zeros((_PCS - 3, _D), jnp.float32)], axis=0)
    acc = None
    for j in range(_NBR):
        th1 = jnp.maximum(_mm(ndn_ref[j, 0], d1p), 0.0)
        a = th1 * sn_ref[j, 0]
        acc = a if acc is None else jnp.maximum(acc, a)
    fm2 = jnp.maximum(cen_ref[0] + acc, 0.0)
    x = _mm(fm2, fc1w_ref[...]) + fc1b_ref[...]
    q_ref[0] = _mm(x, wq_ref[...])
    k_ref[0] = _mm(x, wk_ref[...])
    v_ref[0] = _mm(x, wv_ref[...])
    fm2_ref[0] = fm2


def _conv2(ndn, sn, cen, dir1, fc1_w, fc1br, wq, wk, wv):
    blkmap = lambda b, n: (b, n, 0)
    wmap = lambda b, n: (0, 0)
    od = jax.ShapeDtypeStruct((_B, _N, _D), jnp.float32)
    return pl.pallas_call(
        _conv2_body,
        grid=(_B, _N // _BLK),
        in_specs=[
            pl.BlockSpec((_NBR, 1, _BLK, _PCS), lambda b, n: (0, b, n, 0)),
            pl.BlockSpec((_NBR, 1, _BLK, _D), lambda b, n: (0, b, n, 0)),
            pl.BlockSpec((1, _BLK, _D), blkmap),
            pl.BlockSpec((3, _D), wmap),
            pl.BlockSpec((_D, _D), wmap),
            pl.BlockSpec((1, _D), wmap),
            pl.BlockSpec((_D, _D), wmap),
            pl.BlockSpec((_D, _D), wmap),
            pl.BlockSpec((_D, _D), wmap),
        ],
        out_specs=[pl.BlockSpec((1, _BLK, _D), blkmap)] * 4,
        out_shape=[od, od, od, od],
    )(ndn, sn, cen, dir1, fc1_w, fc1br, wq, wk, wv)


def _attn_body(nbr_ref, vert_ref, q_ref, kk_ref, vv_ref, fm2_ref,
               fcd1_ref, fcd1b_ref, fcd2_ref, fcd2b_ref,
               fcg1_ref, fcg1b_ref, fcg2_ref, fcg2b_ref,
               fc2w_ref, fc2b_ref, out_ref):
    vert = vert_ref[0]
    fd1p = jnp.concatenate(
        [fcd1_ref[...], jnp.zeros((_PC - 3, _D), jnp.float32)], axis=0)
    q = q_ref[0]
    scale = 1.0 / np.sqrt(float(_D))
    zs, ps = [], []
    m = None
    for j in range(_KATT):
        rel = vert - nbr_ref[j, 0]
        h = jnp.maximum(
            _mm(rel, fd1p)
            + fcd1b_ref[...], 0.0)
        pos = _mm(h, fcd2_ref[...]) + fcd2b_ref[...]
        t = q - kk_ref[j, 0] + pos
        g = jnp.maximum(
            _mm(t, fcg1_ref[...])
            + fcg1b_ref[...], 0.0)
        z = (_mm(g, fcg2_ref[...])
             + fcg2b_ref[...]) * scale
        zs.append(z)
        ps.append(pos)
        m = z if m is None else jnp.maximum(m, z)
    es, s = [], None
    for j in range(_KATT):
        e = jnp.exp(zs[j] - m)
        es.append(e)
        s = e if s is None else s + e
    res = None
    for j in range(_KATT):
        c = (es[j] / s) * (vv_ref[j, 0] + ps[j])
        res = c if res is None else res + c
    out_ref[0] = (_mm(res, fc2w_ref[...])
                  + fc2b_ref[...] + fm2_ref[0])


def _attn(nbr, vert_pad, q, kk, vv, fm2,
          fcd1, fcd1br, fcd2, fcd2br, fcg1, fcg1br, fcg2, fcg2br,
          fc2_w, fc2br):
    blkmap = lambda b, n: (b, n, 0)
    wmap = lambda b, n: (0, 0)
    return pl.pallas_call(
        _attn_body,
        grid=(_B, _N // _BLK),
        in_specs=[
            pl.BlockSpec((_TOP, 1, _BLK, _PC), lambda b, n: (0, b, n, 0)),
            pl.BlockSpec((1, _BLK, _PC), blkmap),
            pl.BlockSpec((1, _BLK, _D), blkmap),
            pl.BlockSpec((_KATT, 1, _BLK, _D), lambda b, n: (0, b, n, 0)),
            pl.BlockSpec((_KATT, 1, _BLK, _D), lambda b, n: (0, b, n, 0)),
            pl.BlockSpec((1, _BLK, _D), blkmap),
            pl.BlockSpec((3, _D), wmap),
            pl.BlockSpec((1, _D), wmap),
            pl.BlockSpec((_D, _D), wmap),
            pl.BlockSpec((1, _D), wmap),
            pl.BlockSpec((_D, _D), wmap),
            pl.BlockSpec((1, _D), wmap),
            pl.BlockSpec((_D, _D), wmap),
            pl.BlockSpec((1, _D), wmap),
            pl.BlockSpec((_D, _D), wmap),
            pl.BlockSpec((1, _D), wmap),
        ],
        out_specs=pl.BlockSpec((1, _BLK, _D), blkmap),
        out_shape=jax.ShapeDtypeStruct((_B, _N, _D), jnp.float32),
    )(nbr, vert_pad, q, kk, vv, fm2,
      fcd1, fcd1br, fcd2, fcd2br, fcg1, fcg1br, fcg2, fcg2br,
      fc2_w, fc2br)


def kernel(xyz, dir0, w1, b1, dir1, fc1_w, fc1_b, fc2_w, fc2_b,
           fcd1_w, fcd1_b, fcd2_w, fcd2_b, fcg1_w, fcg1_b, fcg2_w, fcg2_b,
           wq, wk, wv):
    vertices = jnp.transpose(xyz, (0, 2, 1))
    vert_pad = jnp.pad(vertices, ((0, 0), (0, 0), (0, _PC - 3)))

    idx = _topk(xyz, vertices)

    idx21 = jnp.transpose(idx[:, :, :_TOP], (2, 0, 1)).reshape(-1)
    (nbr_flat,) = _sc_gather([vert_pad.reshape(_B * _N, _PC)], idx21)
    nbr = nbr_flat.reshape(_TOP, _B, _N, _PC)

    cen, supp, ndn = _conv1(nbr, vert_pad, dir0,
                            w1.astype(jnp.bfloat16), b1.reshape(1, -1))

    idx20 = jnp.transpose(idx[:, :, 1:_TOP], (2, 0, 1)).reshape(-1)
    (sn_flat,) = _sc_gather([supp.reshape(_B * _N, _D)], idx20)
    sn = sn_flat.reshape(_NBR, _B, _N, _D)

    q, kx, vx, fm2 = _conv2(ndn, sn, cen, dir1,
                            fc1_w.astype(jnp.bfloat16),
                            fc1_b.reshape(1, -1),
                            wq.astype(jnp.bfloat16),
                            wk.astype(jnp.bfloat16),
                            wv.astype(jnp.bfloat16))

    idx16 = jnp.transpose(idx[:, :, :_KATT], (2, 0, 1)).reshape(-1)
    kkf, vvf = _sc_gather(
        [kx.reshape(_B * _N, _D), vx.reshape(_B * _N, _D)], idx16)
    kk = kkf.reshape(_KATT, _B, _N, _D)
    vv = vvf.reshape(_KATT, _B, _N, _D)

    out = _attn(nbr, vert_pad, q, kk, vv, fm2,
                fcd1_w, fcd1_b.reshape(1, -1),
                fcd2_w.astype(jnp.bfloat16), fcd2_b.reshape(1, -1),
                fcg1_w.astype(jnp.bfloat16), fcg1_b.reshape(1, -1),
                fcg2_w.astype(jnp.bfloat16), fcg2_b.reshape(1, -1),
                fc2_w.astype(jnp.bfloat16), fc2_b.reshape(1, -1))
    return jnp.transpose(out, (0, 2, 1))


# speedup vs baseline: 20.8452x; 1.1079x over previous
import functools

import jax
import jax.numpy as jnp
import numpy as np
from jax import lax
from jax.experimental import pallas as pl
from jax.experimental.pallas import tpu as pltpu
from jax.experimental.pallas import tpu_sc as plsc

_B = 4
_N = 2048
_TOP = 21
_NBR = 20
_KATT = 16
_D = 128
_PC = 128
_PCS = 16
_BLK = 256
_IDXW = 32

def _mm(a, b):
    return jnp.dot(a.astype(jnp.bfloat16), b.astype(jnp.bfloat16),
                   preferred_element_type=jnp.float32)


def _r(x):
    return x.astype(jnp.bfloat16).astype(jnp.float32)


_SC_NC, _SC_NS = 2, 16
_NW = _SC_NC * _SC_NS
_CH = 128


def _topk_body(xyzT_ref, vert_ref, idx_ref):
    b = pl.program_id(0)
    va = xyzT_ref[0]
    vb = vert_ref[0]
    va0, va1, va2 = va[0:1, :], va[1:2, :], va[2:3, :]
    vb0, vb1, vb2 = vb[:, 0:1], vb[:, 1:2], vb[:, 2:3]
    inner = (_r(vb0) * _r(va0) + _r(vb1) * _r(va1)
             + _r(vb2) * _r(va2))
    qa = va0 * va0 + va1 * va1 + va2 * va2
    qb = vb0 * vb0 + vb1 * vb1 + vb2 * vb2
    d = -2.0 * inner + qa + qb
    iota = lax.broadcasted_iota(
        jnp.int32, (_BLK, _N), 1).astype(jnp.float32)
    fn = jnp.float32(_N)
    cols = []
    for _ in range(_TOP):
        m = jnp.min(d, axis=1, keepdims=True)
        cand = jnp.where(d == m, iota, fn)
        i = jnp.min(cand, axis=1, keepdims=True)
        cols.append(i)
        d = jnp.where(iota == i, jnp.inf, d)
    idx = jnp.concatenate(cols, axis=1).astype(jnp.int32)
    idx = jnp.concatenate(
        [idx, jnp.zeros((_BLK, _IDXW - _TOP), jnp.int32)], axis=1)
    idx_ref[0] = idx + b * _N


def _topk(xyz, vertices):
    return pl.pallas_call(
        _topk_body,
        grid=(_B, _N // _BLK),
        in_specs=[
            pl.BlockSpec((1, 3, _N), lambda b, n: (b, 0, 0)),
            pl.BlockSpec((1, _BLK, 3), lambda b, n: (b, n, 0)),
        ],
        out_specs=pl.BlockSpec((1, _BLK, _IDXW), lambda b, n: (b, n, 0)),
        out_shape=jax.ShapeDtypeStruct((_B, _N, _IDXW), jnp.int32),
    )(xyz, vertices)


def _sc_gather(tables, idx_flat):
    nt = len(tables)
    rows = idx_flat.shape[0]
    per_w = rows // _NW
    n_ch = per_w // _CH
    mesh = plsc.VectorSubcoreMesh(core_axis_name="c", subcore_axis_name="s")
    out_type = [jax.ShapeDtypeStruct((rows, t.shape[1]), t.dtype)
                for t in tables]
    assert n_ch % 2 == 0
    scratch = ([pltpu.VMEM((per_w,), jnp.int32)]
               + [pltpu.VMEM((_CH, t.shape[1]), jnp.float32)
                  for t in tables for _ in range(2)]
               + [pltpu.SemaphoreType.DMA] * 4)

    @functools.partial(pl.kernel, mesh=mesh, out_type=out_type,
                       scratch_types=scratch)
    def gath(idx_hbm, *rest):
        tabs = rest[:nt]
        outs = rest[nt:2 * nt]
        idx_v = rest[2 * nt]
        bufs = rest[2 * nt + 1:2 * nt + 1 + 2 * nt]
        gsem = rest[-4:-2]
        ssem = rest[-2:]
        wid = lax.axis_index("s") * _SC_NC + lax.axis_index("c")
        base = wid * per_w
        pltpu.sync_copy(idx_hbm.at[pl.ds(base, per_w)], idx_v)

        def body(p, carry):
            gh, sh = [], []
            for s in range(2):
                c = 2 * p + s
                ix = idx_v.at[pl.ds(c * _CH, _CH)]
                gh.append([pltpu.async_copy(tabs[t].at[ix],
                                            bufs[2 * t + s], gsem[s])
                           for t in range(nt)])
            for s in range(2):
                c = 2 * p + s
                off = base + c * _CH
                for t in range(nt):
                    gh[s][t].wait()
                    sh.append(pltpu.async_copy(
                        bufs[2 * t + s], outs[t].at[pl.ds(off, _CH)],
                        ssem[s]))
            for h in sh:
                h.wait()
            return carry

        lax.fori_loop(0, n_ch // 2, body, 0)

    res = gath(idx_flat, *tables)
    return res if isinstance(res, (list, tuple)) else [res]


def _conv1_body(nbr_ref, vert_ref, dir0_ref, w1_ref, b1_ref,
                cen_ref, supp_ref, ndn_ref):
    vert = vert_ref[0]
    d0 = dir0_ref[...]
    n0 = jnp.sqrt(jnp.sum(d0 * d0, axis=0, keepdims=True))
    d0p = jnp.concatenate(
        [d0 / jnp.maximum(n0, 1e-12),
         jnp.zeros((_PC - 3, _D), jnp.float32)], axis=0)
    acc = None
    for j in range(1, _TOP):
        diff = nbr_ref[j, 0] - vert
        nrm = jnp.sqrt(jnp.sum(diff * diff, axis=1, keepdims=True))
        ndn = diff / jnp.maximum(nrm, 1e-12)
        ndn_ref[j - 1, 0] = ndn[:, :_PCS]
        th = jnp.maximum(
            _mm(ndn, d0p), 0.0)
        acc = th if acc is None else jnp.maximum(acc, th)
    fm = jnp.maximum(acc, 0.0)
    fo = _mm(fm, w1_ref[...]) + b1_ref[...]
    cen_ref[0] = fo[:, :_D]
    supp_ref[0] = fo[:, _D:]


def _conv1(nbr, vert_pad, dir0, w1, b1r):
    blkmap = lambda b, n: (b, n, 0)
    return pl.pallas_call(
        _conv1_body,
        grid=(_B, _N // _BLK),
        in_specs=[
            pl.BlockSpec((_TOP, 1, _BLK, _PC), lambda b, n: (0, b, n, 0)),
            pl.BlockSpec((1, _BLK, _PC), blkmap),
            pl.BlockSpec((3, _D), lambda b, n: (0, 0)),
            pl.BlockSpec((_D, 2 * _D), lambda b, n: (0, 0)),
            pl.BlockSpec((1, 2 * _D), lambda b, n: (0, 0)),
        ],
        out_specs=[
            pl.BlockSpec((1, _BLK, _D), blkmap),
            pl.BlockSpec((1, _BLK, _D), blkmap),
            pl.BlockSpec((_NBR, 1, _BLK, _PCS), lambda b, n: (0, b, n, 0)),
        ],
        out_shape=[
            jax.ShapeDtypeStruct((_B, _N, _D), jnp.float32),
            jax.ShapeDtypeStruct((_B, _N, _D), jnp.float32),
            jax.ShapeDtypeStruct((_NBR, _B, _N, _PCS), jnp.float32),
        ],
    )(nbr, vert_pad, dir0, w1, b1r)


def _conv2_body(ndn_ref, sn_ref, cen_ref, dir1_ref, fc1w_ref, fc1b_ref,
                wq_ref, wk_ref, wv_ref, q_ref, k_ref, v_ref, fm2_ref):
    d1 = dir1_ref[...]
    n1 = jnp.sqrt(jnp.sum(d1 * d1, axis=0, keepdims=True))
    d1p = jnp.concatenate(
        [d1 / jnp.maximum(n1, 1e-12),
         jnp.zeros((_PCS - 3, _D), jnp.float32)], axis=0)
    acc = None
    for j in range(_NBR):
        th1 = jnp.maximum(_mm(ndn_ref[j, 0], d1p), 0.0)
        a = th1 * sn_ref[j, 0]
        acc = a if acc is None else jnp.maximum(acc, a)
    fm2 = jnp.maximum(cen_ref[0] + acc, 0.0)
    x = _mm(fm2, fc1w_ref[...]) + fc1b_ref[...]
    q_ref[0] = _mm(x, wq_ref[...])
    k_ref[0] = _mm(x, wk_ref[...])
    v_ref[0] = _mm(x, wv_ref[...])
    fm2_ref[0] = fm2


def _conv2(ndn, sn, cen, dir1, fc1_w, fc1br, wq, wk, wv):
    blkmap = lambda b, n: (b, n, 0)
    wmap = lambda b, n: (0, 0)
    od = jax.ShapeDtypeStruct((_B, _N, _D), jnp.float32)
    return pl.pallas_call(
        _conv2_body,
        grid=(_B, _N // _BLK),
        in_specs=[
            pl.BlockSpec((_NBR, 1, _BLK, _PCS), lambda b, n: (0, b, n, 0)),
            pl.BlockSpec((_NBR, 1, _BLK, _D), lambda b, n: (0, b, n, 0)),
            pl.BlockSpec((1, _BLK, _D), blkmap),
            pl.BlockSpec((3, _D), wmap),
            pl.BlockSpec((_D, _D), wmap),
            pl.BlockSpec((1, _D), wmap),
            pl.BlockSpec((_D, _D), wmap),
            pl.BlockSpec((_D, _D), wmap),
            pl.BlockSpec((_D, _D), wmap),
        ],
        out_specs=[pl.BlockSpec((1, _BLK, _D), blkmap)] * 4,
        out_shape=[od, od, od, od],
    )(ndn, sn, cen, dir1, fc1_w, fc1br, wq, wk, wv)


def _attn_body(nbr_ref, vert_ref, q_ref, kk_ref, vv_ref, fm2_ref,
               fcd1_ref, fcd1b_ref, fcd2_ref, fcd2b_ref,
               fcg1_ref, fcg1b_ref, fcg2_ref, fcg2b_ref,
               fc2w_ref, fc2b_ref, out_ref):
    vert = vert_ref[0]
    fd1p = jnp.concatenate(
        [fcd1_ref[...], jnp.zeros((_PC - 3, _D), jnp.float32)], axis=0)
    q = q_ref[0]
    scale = 1.0 / np.sqrt(float(_D))
    ek = _KATT * _BLK
    knnf = nbr_ref[0:_KATT].reshape(_KATT, _BLK, _PC).reshape(ek, _PC)
    vertf = jnp.broadcast_to(vert[None], (_KATT, _BLK, _PC)).reshape(ek, _PC)
    qf = jnp.broadcast_to(q[None], (_KATT, _BLK, _D)).reshape(ek, _D)
    kkf = kk_ref[...].reshape(_KATT, _BLK, _D).reshape(ek, _D)
    vvf = vv_ref[...].reshape(_KATT, _BLK, _D).reshape(ek, _D)
    rel = vertf - knnf
    h = jnp.maximum(_mm(rel, fd1p) + fcd1b_ref[...], 0.0)
    pos = _mm(h, fcd2_ref[...]) + fcd2b_ref[...]
    t = qf - kkf + pos
    g = jnp.maximum(_mm(t, fcg1_ref[...]) + fcg1b_ref[...], 0.0)
    z = (_mm(g, fcg2_ref[...]) + fcg2b_ref[...]) * scale
    pv = vvf + pos
    zs = [z[j * _BLK:(j + 1) * _BLK] for j in range(_KATT)]
    m = None
    for j in range(_KATT):
        m = zs[j] if m is None else jnp.maximum(m, zs[j])
    es, s = [], None
    for j in range(_KATT):
        e = jnp.exp(zs[j] - m)
        es.append(e)
        s = e if s is None else s + e
    res = None
    for j in range(_KATT):
        c = (es[j] / s) * pv[j * _BLK:(j + 1) * _BLK]
        res = c if res is None else res + c
    out_ref[0] = (_mm(res, fc2w_ref[...])
                  + fc2b_ref[...] + fm2_ref[0])


def _attn(nbr, vert_pad, q, kk, vv, fm2,
          fcd1, fcd1br, fcd2, fcd2br, fcg1, fcg1br, fcg2, fcg2br,
          fc2_w, fc2br):
    blkmap = lambda b, n: (b, n, 0)
    wmap = lambda b, n: (0, 0)
    return pl.pallas_call(
        _attn_body,
        grid=(_B, _N // _BLK),
        in_specs=[
            pl.BlockSpec((_TOP, 1, _BLK, _PC), lambda b, n: (0, b, n, 0)),
            pl.BlockSpec((1, _BLK, _PC), blkmap),
            pl.BlockSpec((1, _BLK, _D), blkmap),
            pl.BlockSpec((_KATT, 1, _BLK, _D), lambda b, n: (0, b, n, 0)),
            pl.BlockSpec((_KATT, 1, _BLK, _D), lambda b, n: (0, b, n, 0)),
            pl.BlockSpec((1, _BLK, _D), blkmap),
            pl.BlockSpec((3, _D), wmap),
            pl.BlockSpec((1, _D), wmap),
            pl.BlockSpec((_D, _D), wmap),
            pl.BlockSpec((1, _D), wmap),
            pl.BlockSpec((_D, _D), wmap),
            pl.BlockSpec((1, _D), wmap),
            pl.BlockSpec((_D, _D), wmap),
            pl.BlockSpec((1, _D), wmap),
            pl.BlockSpec((_D, _D), wmap),
            pl.BlockSpec((1, _D), wmap),
        ],
        out_specs=pl.BlockSpec((1, _BLK, _D), blkmap),
        out_shape=jax.ShapeDtypeStruct((_B, _N, _D), jnp.float32),
    )(nbr, vert_pad, q, kk, vv, fm2,
      fcd1, fcd1br, fcd2, fcd2br, fcg1, fcg1br, fcg2, fcg2br,
      fc2_w, fc2br)


def kernel(xyz, dir0, w1, b1, dir1, fc1_w, fc1_b, fc2_w, fc2_b,
           fcd1_w, fcd1_b, fcd2_w, fcd2_b, fcg1_w, fcg1_b, fcg2_w, fcg2_b,
           wq, wk, wv):
    vertices = jnp.transpose(xyz, (0, 2, 1))
    vert_pad = jnp.pad(vertices, ((0, 0), (0, 0), (0, _PC - 3)))

    idx = _topk(xyz, vertices)

    idx21 = jnp.transpose(idx[:, :, :_TOP], (2, 0, 1)).reshape(-1)
    (nbr_flat,) = _sc_gather([vert_pad.reshape(_B * _N, _PC)], idx21)
    nbr = nbr_flat.reshape(_TOP, _B, _N, _PC)

    cen, supp, ndn = _conv1(nbr, vert_pad, dir0,
                            w1.astype(jnp.bfloat16), b1.reshape(1, -1))

    idx20 = jnp.transpose(idx[:, :, 1:_TOP], (2, 0, 1)).reshape(-1)
    (sn_flat,) = _sc_gather([supp.reshape(_B * _N, _D)], idx20)
    sn = sn_flat.reshape(_NBR, _B, _N, _D)

    q, kx, vx, fm2 = _conv2(ndn, sn, cen, dir1,
                            fc1_w.astype(jnp.bfloat16),
                            fc1_b.reshape(1, -1),
                            wq.astype(jnp.bfloat16),
                            wk.astype(jnp.bfloat16),
                            wv.astype(jnp.bfloat16))

    idx16 = jnp.transpose(idx[:, :, :_KATT], (2, 0, 1)).reshape(-1)
    kkf, vvf = _sc_gather(
        [kx.reshape(_B * _N, _D), vx.reshape(_B * _N, _D)], idx16)
    kk = kkf.reshape(_KATT, _B, _N, _D)
    vv = vvf.reshape(_KATT, _B, _N, _D)

    out = _attn(nbr, vert_pad, q, kk, vv, fm2,
                fcd1_w, fcd1_b.reshape(1, -1),
                fcd2_w.astype(jnp.bfloat16), fcd2_b.reshape(1, -1),
                fcg1_w.astype(jnp.bfloat16), fcg1_b.reshape(1, -1),
                fcg2_w.astype(jnp.bfloat16), fcg2_b.reshape(1, -1),
                fc2_w.astype(jnp.bfloat16), fc2_b.reshape(1, -1))
    return jnp.transpose(out, (0, 2, 1))


# speedup vs baseline: 20.8548x; 1.0005x over previous
import functools

import jax
import jax.numpy as jnp
import numpy as np
from jax import lax
from jax.experimental import pallas as pl
from jax.experimental.pallas import tpu as pltpu
from jax.experimental.pallas import tpu_sc as plsc

_B = 4
_N = 2048
_TOP = 21
_NBR = 20
_KATT = 16
_D = 128
_PC = 128
_PCS = 16
_BLK = 256
_IDXW = 32

def _mm(a, b):
    return jnp.dot(a.astype(jnp.bfloat16), b.astype(jnp.bfloat16),
                   preferred_element_type=jnp.float32)


def _r(x):
    return x.astype(jnp.bfloat16).astype(jnp.float32)


_SC_NC, _SC_NS = 2, 16
_NW = _SC_NC * _SC_NS
_CH = 128


def _topk_body(xyzT_ref, vert_ref, idx_ref):
    b = pl.program_id(0)
    va = xyzT_ref[0]
    vb = vert_ref[0]
    va0, va1, va2 = va[0:1, :], va[1:2, :], va[2:3, :]
    vb0, vb1, vb2 = vb[:, 0:1], vb[:, 1:2], vb[:, 2:3]
    inner = (_r(vb0) * _r(va0) + _r(vb1) * _r(va1)
             + _r(vb2) * _r(va2))
    qa = va0 * va0 + va1 * va1 + va2 * va2
    qb = vb0 * vb0 + vb1 * vb1 + vb2 * vb2
    d = -2.0 * inner + qa + qb
    iota = lax.broadcasted_iota(
        jnp.int32, (_BLK, _N), 1).astype(jnp.float32)
    fn = jnp.float32(_N)
    cols = []
    for _ in range(_TOP):
        m = jnp.min(d, axis=1, keepdims=True)
        cand = jnp.where(d == m, iota, fn)
        i = jnp.min(cand, axis=1, keepdims=True)
        cols.append(i)
        d = jnp.where(iota == i, jnp.inf, d)
    idx = jnp.concatenate(cols, axis=1).astype(jnp.int32)
    idx = jnp.concatenate(
        [idx, jnp.zeros((_BLK, _IDXW - _TOP), jnp.int32)], axis=1)
    idx_ref[0] = idx + b * _N


def _topk(xyz, vertices):
    return pl.pallas_call(
        _topk_body,
        grid=(_B, _N // _BLK),
        in_specs=[
            pl.BlockSpec((1, 3, _N), lambda b, n: (b, 0, 0)),
            pl.BlockSpec((1, _BLK, 3), lambda b, n: (b, n, 0)),
        ],
        out_specs=pl.BlockSpec((1, _BLK, _IDXW), lambda b, n: (b, n, 0)),
        out_shape=jax.ShapeDtypeStruct((_B, _N, _IDXW), jnp.int32),
    )(xyz, vertices)


def _sc_gather(tables, idx_flat):
    nt = len(tables)
    rows = idx_flat.shape[0]
    per_w = rows // _NW
    n_ch = per_w // _CH
    mesh = plsc.VectorSubcoreMesh(core_axis_name="c", subcore_axis_name="s")
    out_type = [jax.ShapeDtypeStruct((rows, t.shape[1]), t.dtype)
                for t in tables]
    assert n_ch % 2 == 0
    scratch = ([pltpu.VMEM((per_w,), jnp.int32)]
               + [pltpu.VMEM((_CH, t.shape[1]), jnp.float32)
                  for t in tables for _ in range(2)]
               + [pltpu.SemaphoreType.DMA] * 4)

    @functools.partial(pl.kernel, mesh=mesh, out_type=out_type,
                       scratch_types=scratch)
    def gath(idx_hbm, *rest):
        tabs = rest[:nt]
        outs = rest[nt:2 * nt]
        idx_v = rest[2 * nt]
        bufs = rest[2 * nt + 1:2 * nt + 1 + 2 * nt]
        gsem = rest[-4:-2]
        ssem = rest[-2:]
        wid = lax.axis_index("s") * _SC_NC + lax.axis_index("c")
        base = wid * per_w
        pltpu.sync_copy(idx_hbm.at[pl.ds(base, per_w)], idx_v)

        def body(p, carry):
            gh, sh = [], []
            for s in range(2):
                c = 2 * p + s
                ix = idx_v.at[pl.ds(c * _CH, _CH)]
                gh.append([pltpu.async_copy(tabs[t].at[ix],
                                            bufs[2 * t + s], gsem[s])
                           for t in range(nt)])
            for s in range(2):
                c = 2 * p + s
                off = base + c * _CH
                for t in range(nt):
                    gh[s][t].wait()
                    sh.append(pltpu.async_copy(
                        bufs[2 * t + s], outs[t].at[pl.ds(off, _CH)],
                        ssem[s]))
            for h in sh:
                h.wait()
            return carry

        lax.fori_loop(0, n_ch // 2, body, 0)

    res = gath(idx_flat, *tables)
    return res if isinstance(res, (list, tuple)) else [res]


def _conv1_body(nbr_ref, vert_ref, dir0_ref, w1_ref, b1_ref,
                cen_ref, supp_ref, ndn_ref):
    vert = vert_ref[0]
    d0 = dir0_ref[...]
    n0 = jnp.sqrt(jnp.sum(d0 * d0, axis=0, keepdims=True))
    d0p = jnp.concatenate(
        [d0 / jnp.maximum(n0, 1e-12),
         jnp.zeros((_PC - 3, _D), jnp.float32)], axis=0)
    en = _NBR * _BLK
    nbrf = nbr_ref[1:_TOP].reshape(_NBR, _BLK, _PC).reshape(en, _PC)
    vertf = jnp.broadcast_to(vert[None], (_NBR, _BLK, _PC)).reshape(en, _PC)
    diff = nbrf - vertf
    nrm = jnp.sqrt(jnp.sum(diff * diff, axis=1, keepdims=True))
    ndn = diff / jnp.maximum(nrm, 1e-12)
    ndn_ref[...] = ndn[:, :_PCS].reshape(_NBR, 1, _BLK, _PCS)
    th = jnp.maximum(_mm(ndn, d0p), 0.0)
    acc = None
    for j in range(_NBR):
        acc = (th[j * _BLK:(j + 1) * _BLK] if acc is None
               else jnp.maximum(acc, th[j * _BLK:(j + 1) * _BLK]))
    fm = jnp.maximum(acc, 0.0)
    fo = _mm(fm, w1_ref[...]) + b1_ref[...]
    cen_ref[0] = fo[:, :_D]
    supp_ref[0] = fo[:, _D:]


def _conv1(nbr, vert_pad, dir0, w1, b1r):
    blkmap = lambda b, n: (b, n, 0)
    return pl.pallas_call(
        _conv1_body,
        grid=(_B, _N // _BLK),
        in_specs=[
            pl.BlockSpec((_TOP, 1, _BLK, _PC), lambda b, n: (0, b, n, 0)),
            pl.BlockSpec((1, _BLK, _PC), blkmap),
            pl.BlockSpec((3, _D), lambda b, n: (0, 0)),
            pl.BlockSpec((_D, 2 * _D), lambda b, n: (0, 0)),
            pl.BlockSpec((1, 2 * _D), lambda b, n: (0, 0)),
        ],
        out_specs=[
            pl.BlockSpec((1, _BLK, _D), blkmap),
            pl.BlockSpec((1, _BLK, _D), blkmap),
            pl.BlockSpec((_NBR, 1, _BLK, _PCS), lambda b, n: (0, b, n, 0)),
        ],
        out_shape=[
            jax.ShapeDtypeStruct((_B, _N, _D), jnp.float32),
            jax.ShapeDtypeStruct((_B, _N, _D), jnp.float32),
            jax.ShapeDtypeStruct((_NBR, _B, _N, _PCS), jnp.float32),
        ],
    )(nbr, vert_pad, dir0, w1, b1r)


def _conv2_body(ndn_ref, sn_ref, cen_ref, dir1_ref, fc1w_ref, fc1b_ref,
                wq_ref, wk_ref, wv_ref, q_ref, k_ref, v_ref, fm2_ref):
    d1 = dir1_ref[...]
    n1 = jnp.sqrt(jnp.sum(d1 * d1, axis=0, keepdims=True))
    d1p = jnp.concatenate(
        [d1 / jnp.maximum(n1, 1e-12),
         jnp.zeros((_PCS - 3, _D), jnp.float32)], axis=0)
    en = _NBR * _BLK
    ndnf = ndn_ref[...].reshape(_NBR, _BLK, _PCS).reshape(en, _PCS)
    snf = sn_ref[...].reshape(_NBR, _BLK, _D).reshape(en, _D)
    a = jnp.maximum(_mm(ndnf, d1p), 0.0) * snf
    acc = None
    for j in range(_NBR):
        acc = (a[j * _BLK:(j + 1) * _BLK] if acc is None
               else jnp.maximum(acc, a[j * _BLK:(j + 1) * _BLK]))
    fm2 = jnp.maximum(cen_ref[0] + acc, 0.0)
    x = _mm(fm2, fc1w_ref[...]) + fc1b_ref[...]
    q_ref[0] = _mm(x, wq_ref[...])
    k_ref[0] = _mm(x, wk_ref[...])
    v_ref[0] = _mm(x, wv_ref[...])
    fm2_ref[0] = fm2


def _conv2(ndn, sn, cen, dir1, fc1_w, fc1br, wq, wk, wv):
    blkmap = lambda b, n: (b, n, 0)
    wmap = lambda b, n: (0, 0)
    od = jax.ShapeDtypeStruct((_B, _N, _D), jnp.float32)
    return pl.pallas_call(
        _conv2_body,
        grid=(_B, _N // _BLK),
        in_specs=[
            pl.BlockSpec((_NBR, 1, _BLK, _PCS), lambda b, n: (0, b, n, 0)),
            pl.BlockSpec((_NBR, 1, _BLK, _D), lambda b, n: (0, b, n, 0)),
            pl.BlockSpec((1, _BLK, _D), blkmap),
            pl.BlockSpec((3, _D), wmap),
            pl.BlockSpec((_D, _D), wmap),
            pl.BlockSpec((1, _D), wmap),
            pl.BlockSpec((_D, _D), wmap),
            pl.BlockSpec((_D, _D), wmap),
            pl.BlockSpec((_D, _D), wmap),
        ],
        out_specs=[pl.BlockSpec((1, _BLK, _D), blkmap)] * 4,
        out_shape=[od, od, od, od],
    )(ndn, sn, cen, dir1, fc1_w, fc1br, wq, wk, wv)


def _attn_body(nbr_ref, vert_ref, q_ref, kk_ref, vv_ref, fm2_ref,
               fcd1_ref, fcd1b_ref, fcd2_ref, fcd2b_ref,
               fcg1_ref, fcg1b_ref, fcg2_ref, fcg2b_ref,
               fc2w_ref, fc2b_ref, out_ref):
    vert = vert_ref[0]
    fd1p = jnp.concatenate(
        [fcd1_ref[...], jnp.zeros((_PC - 3, _D), jnp.float32)], axis=0)
    q = q_ref[0]
    scale = 1.0 / np.sqrt(float(_D))
    ek = _KATT * _BLK
    knnf = nbr_ref[0:_KATT].reshape(_KATT, _BLK, _PC).reshape(ek, _PC)
    vertf = jnp.broadcast_to(vert[None], (_KATT, _BLK, _PC)).reshape(ek, _PC)
    qf = jnp.broadcast_to(q[None], (_KATT, _BLK, _D)).reshape(ek, _D)
    kkf = kk_ref[...].reshape(_KATT, _BLK, _D).reshape(ek, _D)
    vvf = vv_ref[...].reshape(_KATT, _BLK, _D).reshape(ek, _D)
    rel = vertf - knnf
    h = jnp.maximum(_mm(rel, fd1p) + fcd1b_ref[...], 0.0)
    pos = _mm(h, fcd2_ref[...]) + fcd2b_ref[...]
    t = qf - kkf + pos
    g = jnp.maximum(_mm(t, fcg1_ref[...]) + fcg1b_ref[...], 0.0)
    z = (_mm(g, fcg2_ref[...]) + fcg2b_ref[...]) * scale
    pv = vvf + pos
    zs = [z[j * _BLK:(j + 1) * _BLK] for j in range(_KATT)]
    m = None
    for j in range(_KATT):
        m = zs[j] if m is None else jnp.maximum(m, zs[j])
    es, s = [], None
    for j in range(_KATT):
        e = jnp.exp(zs[j] - m)
        es.append(e)
        s = e if s is None else s + e
    res = None
    for j in range(_KATT):
        c = (es[j] / s) * pv[j * _BLK:(j + 1) * _BLK]
        res = c if res is None else res + c
    out_ref[0] = (_mm(res, fc2w_ref[...])
                  + fc2b_ref[...] + fm2_ref[0])


def _attn(nbr, vert_pad, q, kk, vv, fm2,
          fcd1, fcd1br, fcd2, fcd2br, fcg1, fcg1br, fcg2, fcg2br,
          fc2_w, fc2br):
    blkmap = lambda b, n: (b, n, 0)
    wmap = lambda b, n: (0, 0)
    return pl.pallas_call(
        _attn_body,
        grid=(_B, _N // _BLK),
        in_specs=[
            pl.BlockSpec((_TOP, 1, _BLK, _PC), lambda b, n: (0, b, n, 0)),
            pl.BlockSpec((1, _BLK, _PC), blkmap),
            pl.BlockSpec((1, _BLK, _D), blkmap),
            pl.BlockSpec((_KATT, 1, _BLK, _D), lambda b, n: (0, b, n, 0)),
            pl.BlockSpec((_KATT, 1, _BLK, _D), lambda b, n: (0, b, n, 0)),
            pl.BlockSpec((1, _BLK, _D), blkmap),
            pl.BlockSpec((3, _D), wmap),
            pl.BlockSpec((1, _D), wmap),
            pl.BlockSpec((_D, _D), wmap),
            pl.BlockSpec((1, _D), wmap),
            pl.BlockSpec((_D, _D), wmap),
            pl.BlockSpec((1, _D), wmap),
            pl.BlockSpec((_D, _D), wmap),
            pl.BlockSpec((1, _D), wmap),
            pl.BlockSpec((_D, _D), wmap),
            pl.BlockSpec((1, _D), wmap),
        ],
        out_specs=pl.BlockSpec((1, _BLK, _D), blkmap),
        out_shape=jax.ShapeDtypeStruct((_B, _N, _D), jnp.float32),
    )(nbr, vert_pad, q, kk, vv, fm2,
      fcd1, fcd1br, fcd2, fcd2br, fcg1, fcg1br, fcg2, fcg2br,
      fc2_w, fc2br)


def kernel(xyz, dir0, w1, b1, dir1, fc1_w, fc1_b, fc2_w, fc2_b,
           fcd1_w, fcd1_b, fcd2_w, fcd2_b, fcg1_w, fcg1_b, fcg2_w, fcg2_b,
           wq, wk, wv):
    vertices = jnp.transpose(xyz, (0, 2, 1))
    vert_pad = jnp.pad(vertices, ((0, 0), (0, 0), (0, _PC - 3)))

    idx = _topk(xyz, vertices)

    idx21 = jnp.transpose(idx[:, :, :_TOP], (2, 0, 1)).reshape(-1)
    (nbr_flat,) = _sc_gather([vert_pad.reshape(_B * _N, _PC)], idx21)
    nbr = nbr_flat.reshape(_TOP, _B, _N, _PC)

    cen, supp, ndn = _conv1(nbr, vert_pad, dir0,
                            w1.astype(jnp.bfloat16), b1.reshape(1, -1))

    idx20 = jnp.transpose(idx[:, :, 1:_TOP], (2, 0, 1)).reshape(-1)
    (sn_flat,) = _sc_gather([supp.reshape(_B * _N, _D)], idx20)
    sn = sn_flat.reshape(_NBR, _B, _N, _D)

    q, kx, vx, fm2 = _conv2(ndn, sn, cen, dir1,
                            fc1_w.astype(jnp.bfloat16),
                            fc1_b.reshape(1, -1),
                            wq.astype(jnp.bfloat16),
                            wk.astype(jnp.bfloat16),
                            wv.astype(jnp.bfloat16))

    idx16 = jnp.transpose(idx[:, :, :_KATT], (2, 0, 1)).reshape(-1)
    kkf, vvf = _sc_gather(
        [kx.reshape(_B * _N, _D), vx.reshape(_B * _N, _D)], idx16)
    kk = kkf.reshape(_KATT, _B, _N, _D)
    vv = vvf.reshape(_KATT, _B, _N, _D)

    out = _attn(nbr, vert_pad, q, kk, vv, fm2,
                fcd1_w, fcd1_b.reshape(1, -1),
                fcd2_w.astype(jnp.bfloat16), fcd2_b.reshape(1, -1),
                fcg1_w.astype(jnp.bfloat16), fcg1_b.reshape(1, -1),
                fcg2_w.astype(jnp.bfloat16), fcg2_b.reshape(1, -1),
                fc2_w.astype(jnp.bfloat16), fc2_b.reshape(1, -1))
    return jnp.transpose(out, (0, 2, 1))


# speedup vs baseline: 21.0329x; 1.0085x over previous
import functools

import jax
import jax.numpy as jnp
import numpy as np
from jax import lax
from jax.experimental import pallas as pl
from jax.experimental.pallas import tpu as pltpu
from jax.experimental.pallas import tpu_sc as plsc

_B = 4
_N = 2048
_TOP = 21
_NBR = 20
_KATT = 16
_D = 128
_PC = 128
_PCS = 16
_BLK = 256
_IDXW = 32

def _mm(a, b):
    return jnp.dot(a.astype(jnp.bfloat16), b.astype(jnp.bfloat16),
                   preferred_element_type=jnp.float32)


def _r(x):
    return x.astype(jnp.bfloat16).astype(jnp.float32)


_SC_NC, _SC_NS = 2, 16
_NW = _SC_NC * _SC_NS
_CH = 128


def _topk_body(xyzT_ref, vert_ref, idx_ref):
    b = pl.program_id(0)
    va = xyzT_ref[0]
    vb = vert_ref[0]
    va0, va1, va2 = va[0:1, :], va[1:2, :], va[2:3, :]
    vb0, vb1, vb2 = vb[:, 0:1], vb[:, 1:2], vb[:, 2:3]
    inner = (_r(vb0) * _r(va0) + _r(vb1) * _r(va1)
             + _r(vb2) * _r(va2))
    qa = va0 * va0 + va1 * va1 + va2 * va2
    qb = vb0 * vb0 + vb1 * vb1 + vb2 * vb2
    d = -2.0 * inner + qa + qb
    iota = lax.broadcasted_iota(
        jnp.int32, (_BLK, _N), 1).astype(jnp.float32)
    fn = jnp.float32(_N)
    cols = []
    for _ in range(_TOP):
        m = jnp.min(d, axis=1, keepdims=True)
        cand = jnp.where(d == m, iota, fn)
        i = jnp.min(cand, axis=1, keepdims=True)
        cols.append(i)
        d = jnp.where(iota == i, jnp.inf, d)
    idx = jnp.concatenate(cols, axis=1).astype(jnp.int32)
    idx = jnp.concatenate(
        [idx, jnp.zeros((_BLK, _IDXW - _TOP), jnp.int32)], axis=1)
    idx_ref[0] = idx + b * _N


def _topk(xyz, vertices):
    return pl.pallas_call(
        _topk_body,
        grid=(_B, _N // _BLK),
        in_specs=[
            pl.BlockSpec((1, 3, _N), lambda b, n: (b, 0, 0)),
            pl.BlockSpec((1, _BLK, 3), lambda b, n: (b, n, 0)),
        ],
        out_specs=pl.BlockSpec((1, _BLK, _IDXW), lambda b, n: (b, n, 0)),
        out_shape=jax.ShapeDtypeStruct((_B, _N, _IDXW), jnp.int32),
    )(xyz, vertices)


def _sc_gather(tables, idx_flat, ns=2):
    nt = len(tables)
    rows = idx_flat.shape[0]
    per_w = rows // _NW
    n_ch = per_w // _CH
    mesh = plsc.VectorSubcoreMesh(core_axis_name="c", subcore_axis_name="s")
    out_type = [jax.ShapeDtypeStruct((rows, t.shape[1]), t.dtype)
                for t in tables]
    assert n_ch % ns == 0
    scratch = ([pltpu.VMEM((per_w,), jnp.int32)]
               + [pltpu.VMEM((_CH, t.shape[1]), jnp.float32)
                  for t in tables for _ in range(ns)]
               + [pltpu.SemaphoreType.DMA] * (2 * ns))

    @functools.partial(pl.kernel, mesh=mesh, out_type=out_type,
                       scratch_types=scratch)
    def gath(idx_hbm, *rest):
        tabs = rest[:nt]
        outs = rest[nt:2 * nt]
        idx_v = rest[2 * nt]
        bufs = rest[2 * nt + 1:2 * nt + 1 + ns * nt]
        gsem = rest[-2 * ns:-ns]
        ssem = rest[-ns:]
        wid = lax.axis_index("s") * _SC_NC + lax.axis_index("c")
        base = wid * per_w
        pltpu.sync_copy(idx_hbm.at[pl.ds(base, per_w)], idx_v)

        def body(p, carry):
            gh, sh = [], []
            for s in range(ns):
                c = ns * p + s
                ix = idx_v.at[pl.ds(c * _CH, _CH)]
                gh.append([pltpu.async_copy(tabs[t].at[ix],
                                            bufs[ns * t + s], gsem[s])
                           for t in range(nt)])
            for s in range(ns):
                c = ns * p + s
                off = base + c * _CH
                for t in range(nt):
                    gh[s][t].wait()
                    sh.append(pltpu.async_copy(
                        bufs[ns * t + s], outs[t].at[pl.ds(off, _CH)],
                        ssem[s]))
            for h in sh:
                h.wait()
            return carry

        lax.fori_loop(0, n_ch // ns, body, 0)

    res = gath(idx_flat, *tables)
    return res if isinstance(res, (list, tuple)) else [res]


def _conv1_body(nbr_ref, vert_ref, dir0_ref, w1_ref, b1_ref,
                cen_ref, supp_ref, ndn_ref):
    vert = vert_ref[0]
    d0 = dir0_ref[...]
    n0 = jnp.sqrt(jnp.sum(d0 * d0, axis=0, keepdims=True))
    d0p = jnp.concatenate(
        [d0 / jnp.maximum(n0, 1e-12),
         jnp.zeros((_PC - 3, _D), jnp.float32)], axis=0)
    en = _NBR * _BLK
    nbrf = nbr_ref[1:_TOP].reshape(_NBR, _BLK, _PC).reshape(en, _PC)
    vertf = jnp.broadcast_to(vert[None], (_NBR, _BLK, _PC)).reshape(en, _PC)
    diff = nbrf - vertf
    nrm = jnp.sqrt(jnp.sum(diff * diff, axis=1, keepdims=True))
    ndn = diff / jnp.maximum(nrm, 1e-12)
    ndn_ref[...] = ndn[:, :_PCS].reshape(_NBR, 1, _BLK, _PCS)
    th = jnp.maximum(_mm(ndn, d0p), 0.0)
    acc = None
    for j in range(_NBR):
        acc = (th[j * _BLK:(j + 1) * _BLK] if acc is None
               else jnp.maximum(acc, th[j * _BLK:(j + 1) * _BLK]))
    fm = jnp.maximum(acc, 0.0)
    fo = _mm(fm, w1_ref[...]) + b1_ref[...]
    cen_ref[0] = fo[:, :_D]
    supp_ref[0] = fo[:, _D:]


def _conv1(nbr, vert_pad, dir0, w1, b1r):
    blkmap = lambda b, n: (b, n, 0)
    return pl.pallas_call(
        _conv1_body,
        grid=(_B, _N // _BLK),
        in_specs=[
            pl.BlockSpec((_TOP, 1, _BLK, _PC), lambda b, n: (0, b, n, 0)),
            pl.BlockSpec((1, _BLK, _PC), blkmap),
            pl.BlockSpec((3, _D), lambda b, n: (0, 0)),
            pl.BlockSpec((_D, 2 * _D), lambda b, n: (0, 0)),
            pl.BlockSpec((1, 2 * _D), lambda b, n: (0, 0)),
        ],
        out_specs=[
            pl.BlockSpec((1, _BLK, _D), blkmap),
            pl.BlockSpec((1, _BLK, _D), blkmap),
            pl.BlockSpec((_NBR, 1, _BLK, _PCS), lambda b, n: (0, b, n, 0)),
        ],
        out_shape=[
            jax.ShapeDtypeStruct((_B, _N, _D), jnp.float32),
            jax.ShapeDtypeStruct((_B, _N, _D), jnp.float32),
            jax.ShapeDtypeStruct((_NBR, _B, _N, _PCS), jnp.float32),
        ],
    )(nbr, vert_pad, dir0, w1, b1r)


def _conv2_body(ndn_ref, sn_ref, cen_ref, dir1_ref, fc1w_ref, fc1b_ref,
                wq_ref, wk_ref, wv_ref, q_ref, kv_ref, fm2_ref):
    d1 = dir1_ref[...]
    n1 = jnp.sqrt(jnp.sum(d1 * d1, axis=0, keepdims=True))
    d1p = jnp.concatenate(
        [d1 / jnp.maximum(n1, 1e-12),
         jnp.zeros((_PCS - 3, _D), jnp.float32)], axis=0)
    en = _NBR * _BLK
    ndnf = ndn_ref[...].reshape(_NBR, _BLK, _PCS).reshape(en, _PCS)
    snf = sn_ref[...].reshape(_NBR, _BLK, _D).reshape(en, _D)
    a = jnp.maximum(_mm(ndnf, d1p), 0.0) * snf
    acc = None
    for j in range(_NBR):
        acc = (a[j * _BLK:(j + 1) * _BLK] if acc is None
               else jnp.maximum(acc, a[j * _BLK:(j + 1) * _BLK]))
    fm2 = jnp.maximum(cen_ref[0] + acc, 0.0)
    x = _mm(fm2, fc1w_ref[...]) + fc1b_ref[...]
    q_ref[0] = _mm(x, wq_ref[...])
    kv_ref[0] = jnp.concatenate(
        [_mm(x, wk_ref[...]), _mm(x, wv_ref[...])], axis=1)
    fm2_ref[0] = fm2


def _conv2(ndn, sn, cen, dir1, fc1_w, fc1br, wq, wk, wv):
    blkmap = lambda b, n: (b, n, 0)
    wmap = lambda b, n: (0, 0)
    od = jax.ShapeDtypeStruct((_B, _N, _D), jnp.float32)
    return pl.pallas_call(
        _conv2_body,
        grid=(_B, _N // _BLK),
        in_specs=[
            pl.BlockSpec((_NBR, 1, _BLK, _PCS), lambda b, n: (0, b, n, 0)),
            pl.BlockSpec((_NBR, 1, _BLK, _D), lambda b, n: (0, b, n, 0)),
            pl.BlockSpec((1, _BLK, _D), blkmap),
            pl.BlockSpec((3, _D), wmap),
            pl.BlockSpec((_D, _D), wmap),
            pl.BlockSpec((1, _D), wmap),
            pl.BlockSpec((_D, _D), wmap),
            pl.BlockSpec((_D, _D), wmap),
            pl.BlockSpec((_D, _D), wmap),
        ],
        out_specs=[pl.BlockSpec((1, _BLK, _D), blkmap),
                   pl.BlockSpec((1, _BLK, 2 * _D), blkmap),
                   pl.BlockSpec((1, _BLK, _D), blkmap)],
        out_shape=[od, jax.ShapeDtypeStruct((_B, _N, 2 * _D), jnp.float32),
                   od],
    )(ndn, sn, cen, dir1, fc1_w, fc1br, wq, wk, wv)


def _attn_body(nbr_ref, vert_ref, q_ref, kv_ref, fm2_ref,
               fcd1_ref, fcd1b_ref, fcd2_ref, fcd2b_ref,
               fcg1_ref, fcg1b_ref, fcg2_ref, fcg2b_ref,
               fc2w_ref, fc2b_ref, out_ref):
    vert = vert_ref[0]
    fd1p = jnp.concatenate(
        [fcd1_ref[...], jnp.zeros((_PC - 3, _D), jnp.float32)], axis=0)
    q = q_ref[0]
    scale = 1.0 / np.sqrt(float(_D))
    ek = _KATT * _BLK
    knnf = nbr_ref[0:_KATT].reshape(_KATT, _BLK, _PC).reshape(ek, _PC)
    vertf = jnp.broadcast_to(vert[None], (_KATT, _BLK, _PC)).reshape(ek, _PC)
    qf = jnp.broadcast_to(q[None], (_KATT, _BLK, _D)).reshape(ek, _D)
    kvf = kv_ref[...].reshape(_KATT, _BLK, 2 * _D).reshape(ek, 2 * _D)
    kkf = kvf[:, :_D]
    vvf = kvf[:, _D:]
    rel = vertf - knnf
    h = jnp.maximum(_mm(rel, fd1p) + fcd1b_ref[...], 0.0)
    pos = _mm(h, fcd2_ref[...]) + fcd2b_ref[...]
    t = qf - kkf + pos
    g = jnp.maximum(_mm(t, fcg1_ref[...]) + fcg1b_ref[...], 0.0)
    z = (_mm(g, fcg2_ref[...]) + fcg2b_ref[...]) * scale
    pv = vvf + pos
    zs = [z[j * _BLK:(j + 1) * _BLK] for j in range(_KATT)]
    m = None
    for j in range(_KATT):
        m = zs[j] if m is None else jnp.maximum(m, zs[j])
    es, s = [], None
    for j in range(_KATT):
        e = jnp.exp(zs[j] - m)
        es.append(e)
        s = e if s is None else s + e
    res = None
    for j in range(_KATT):
        c = (es[j] / s) * pv[j * _BLK:(j + 1) * _BLK]
        res = c if res is None else res + c
    out_ref[0] = (_mm(res, fc2w_ref[...])
                  + fc2b_ref[...] + fm2_ref[0])


def _attn(nbr, vert_pad, q, kv, fm2,
          fcd1, fcd1br, fcd2, fcd2br, fcg1, fcg1br, fcg2, fcg2br,
          fc2_w, fc2br):
    blkmap = lambda b, n: (b, n, 0)
    wmap = lambda b, n: (0, 0)
    return pl.pallas_call(
        _attn_body,
        grid=(_B, _N // _BLK),
        in_specs=[
            pl.BlockSpec((_TOP, 1, _BLK, _PC), lambda b, n: (0, b, n, 0)),
            pl.BlockSpec((1, _BLK, _PC), blkmap),
            pl.BlockSpec((1, _BLK, _D), blkmap),
            pl.BlockSpec((_KATT, 1, _BLK, 2 * _D),
                         lambda b, n: (0, b, n, 0)),
            pl.BlockSpec((1, _BLK, _D), blkmap),
            pl.BlockSpec((3, _D), wmap),
            pl.BlockSpec((1, _D), wmap),
            pl.BlockSpec((_D, _D), wmap),
            pl.BlockSpec((1, _D), wmap),
            pl.BlockSpec((_D, _D), wmap),
            pl.BlockSpec((1, _D), wmap),
            pl.BlockSpec((_D, _D), wmap),
            pl.BlockSpec((1, _D), wmap),
            pl.BlockSpec((_D, _D), wmap),
            pl.BlockSpec((1, _D), wmap),
        ],
        out_specs=pl.BlockSpec((1, _BLK, _D), blkmap),
        out_shape=jax.ShapeDtypeStruct((_B, _N, _D), jnp.float32),
    )(nbr, vert_pad, q, kv, fm2,
      fcd1, fcd1br, fcd2, fcd2br, fcg1, fcg1br, fcg2, fcg2br,
      fc2_w, fc2br)


def kernel(xyz, dir0, w1, b1, dir1, fc1_w, fc1_b, fc2_w, fc2_b,
           fcd1_w, fcd1_b, fcd2_w, fcd2_b, fcg1_w, fcg1_b, fcg2_w, fcg2_b,
           wq, wk, wv):
    vertices = jnp.transpose(xyz, (0, 2, 1))
    vert_pad = jnp.pad(vertices, ((0, 0), (0, 0), (0, _PC - 3)))

    idx = _topk(xyz, vertices)

    idx21 = jnp.transpose(idx[:, :, :_TOP], (2, 0, 1)).reshape(-1)
    (nbr_flat,) = _sc_gather([vert_pad.reshape(_B * _N, _PC)], idx21)
    nbr = nbr_flat.reshape(_TOP, _B, _N, _PC)

    cen, supp, ndn = _conv1(nbr, vert_pad, dir0,
                            w1.astype(jnp.bfloat16), b1.reshape(1, -1))

    idx20 = jnp.transpose(idx[:, :, 1:_TOP], (2, 0, 1)).reshape(-1)
    (sn_flat,) = _sc_gather([supp.reshape(_B * _N, _D)], idx20, ns=4)
    sn = sn_flat.reshape(_NBR, _B, _N, _D)

    q, kvx, fm2 = _conv2(ndn, sn, cen, dir1,
                            fc1_w.astype(jnp.bfloat16),
                            fc1_b.reshape(1, -1),
                            wq.astype(jnp.bfloat16),
                            wk.astype(jnp.bfloat16),
                            wv.astype(jnp.bfloat16))

    idx16 = jnp.transpose(idx[:, :, :_KATT], (2, 0, 1)).reshape(-1)
    (kvf,) = _sc_gather([kvx.reshape(_B * _N, 2 * _D)], idx16, ns=2)
    kv = kvf.reshape(_KATT, _B, _N, 2 * _D)

    out = _attn(nbr, vert_pad, q, kv, fm2,
                fcd1_w, fcd1_b.reshape(1, -1),
                fcd2_w.astype(jnp.bfloat16), fcd2_b.reshape(1, -1),
                fcg1_w.astype(jnp.bfloat16), fcg1_b.reshape(1, -1),
                fcg2_w.astype(jnp.bfloat16), fcg2_b.reshape(1, -1),
                fc2_w.astype(jnp.bfloat16), fc2_b.reshape(1, -1))
    return jnp.transpose(out, (0, 2, 1))


# speedup vs baseline: 22.6385x; 1.0763x over previous
import functools

import jax
import jax.numpy as jnp
import numpy as np
from jax import lax
from jax.experimental import pallas as pl
from jax.experimental.pallas import tpu as pltpu
from jax.experimental.pallas import tpu_sc as plsc

_B = 4
_N = 2048
_TOP = 21
_NBR = 20
_KATT = 16
_D = 128
_PC = 128
_PCS = 16
_BLK = 256
_IDXW = 32

def _mm(a, b):
    return jnp.dot(a.astype(jnp.bfloat16), b.astype(jnp.bfloat16),
                   preferred_element_type=jnp.float32)


def _r(x):
    return x.astype(jnp.bfloat16).astype(jnp.float32)


_SC_NC, _SC_NS = 2, 16
_NW = _SC_NC * _SC_NS
_CH = 128


def _topk_body(xyzT_ref, vert_ref, idx_ref):
    b = pl.program_id(0)
    va = xyzT_ref[0]
    vb = vert_ref[0]
    va0, va1, va2 = va[0:1, :], va[1:2, :], va[2:3, :]
    vb0, vb1, vb2 = vb[:, 0:1], vb[:, 1:2], vb[:, 2:3]
    inner = (_r(vb0) * _r(va0) + _r(vb1) * _r(va1)
             + _r(vb2) * _r(va2))
    qa = va0 * va0 + va1 * va1 + va2 * va2
    qb = vb0 * vb0 + vb1 * vb1 + vb2 * vb2
    d = -2.0 * inner + qa + qb
    iota = lax.broadcasted_iota(
        jnp.int32, (_BLK, _N), 1).astype(jnp.float32)
    fn = jnp.float32(_N)
    cols = []
    for _ in range(_TOP):
        m = jnp.min(d, axis=1, keepdims=True)
        cand = jnp.where(d == m, iota, fn)
        i = jnp.min(cand, axis=1, keepdims=True)
        cols.append(i)
        d = jnp.where(iota == i, jnp.inf, d)
    idx = jnp.concatenate(cols, axis=1).astype(jnp.int32)
    idx = jnp.concatenate(
        [idx, jnp.zeros((_BLK, _IDXW - _TOP), jnp.int32)], axis=1)
    idx_ref[0] = idx + b * _N


def _topk(xyz, vertices, nb):
    return pl.pallas_call(
        _topk_body,
        grid=(nb, _N // _BLK),
        in_specs=[
            pl.BlockSpec((1, 3, _N), lambda b, n: (b, 0, 0)),
            pl.BlockSpec((1, _BLK, 3), lambda b, n: (b, n, 0)),
        ],
        out_specs=pl.BlockSpec((1, _BLK, _IDXW), lambda b, n: (b, n, 0)),
        out_shape=jax.ShapeDtypeStruct((nb, _N, _IDXW), jnp.int32),
    )(xyz, vertices)


def _sc_gather(tables, idx_flat, ns=2):
    nt = len(tables)
    rows = idx_flat.shape[0]
    per_w = rows // _NW
    n_ch = per_w // _CH
    mesh = plsc.VectorSubcoreMesh(core_axis_name="c", subcore_axis_name="s")
    out_type = [jax.ShapeDtypeStruct((rows, t.shape[1]), t.dtype)
                for t in tables]
    assert n_ch % ns == 0
    scratch = ([pltpu.VMEM((per_w,), jnp.int32)]
               + [pltpu.VMEM((_CH, t.shape[1]), jnp.float32)
                  for t in tables for _ in range(ns)]
               + [pltpu.SemaphoreType.DMA] * (2 * ns))

    @functools.partial(pl.kernel, mesh=mesh, out_type=out_type,
                       scratch_types=scratch)
    def gath(idx_hbm, *rest):
        tabs = rest[:nt]
        outs = rest[nt:2 * nt]
        idx_v = rest[2 * nt]
        bufs = rest[2 * nt + 1:2 * nt + 1 + ns * nt]
        gsem = rest[-2 * ns:-ns]
        ssem = rest[-ns:]
        wid = lax.axis_index("s") * _SC_NC + lax.axis_index("c")
        base = wid * per_w
        pltpu.sync_copy(idx_hbm.at[pl.ds(base, per_w)], idx_v)

        def body(p, carry):
            gh, sh = [], []
            for s in range(ns):
                c = ns * p + s
                ix = idx_v.at[pl.ds(c * _CH, _CH)]
                gh.append([pltpu.async_copy(tabs[t].at[ix],
                                            bufs[ns * t + s], gsem[s])
                           for t in range(nt)])
            for s in range(ns):
                c = ns * p + s
                off = base + c * _CH
                for t in range(nt):
                    gh[s][t].wait()
                    sh.append(pltpu.async_copy(
                        bufs[ns * t + s], outs[t].at[pl.ds(off, _CH)],
                        ssem[s]))
            for h in sh:
                h.wait()
            return carry

        lax.fori_loop(0, n_ch // ns, body, 0)

    res = gath(idx_flat, *tables)
    return res if isinstance(res, (list, tuple)) else [res]


def _conv1_body(nbr_ref, vert_ref, dir0_ref, w1_ref, b1_ref,
                cen_ref, supp_ref, ndn_ref):
    vert = vert_ref[0]
    d0 = dir0_ref[...]
    n0 = jnp.sqrt(jnp.sum(d0 * d0, axis=0, keepdims=True))
    d0p = jnp.concatenate(
        [d0 / jnp.maximum(n0, 1e-12),
         jnp.zeros((_PC - 3, _D), jnp.float32)], axis=0)
    en = _NBR * _BLK
    nbrf = nbr_ref[1:_TOP].reshape(_NBR, _BLK, _PC).reshape(en, _PC)
    vertf = jnp.broadcast_to(vert[None], (_NBR, _BLK, _PC)).reshape(en, _PC)
    diff = nbrf - vertf
    nrm = jnp.sqrt(jnp.sum(diff * diff, axis=1, keepdims=True))
    ndn = diff / jnp.maximum(nrm, 1e-12)
    ndn_ref[...] = ndn[:, :_PCS].reshape(_NBR, 1, _BLK, _PCS)
    th = jnp.maximum(_mm(ndn, d0p), 0.0)
    acc = None
    for j in range(_NBR):
        acc = (th[j * _BLK:(j + 1) * _BLK] if acc is None
               else jnp.maximum(acc, th[j * _BLK:(j + 1) * _BLK]))
    fm = jnp.maximum(acc, 0.0)
    fo = _mm(fm, w1_ref[...]) + b1_ref[...]
    cen_ref[0] = fo[:, :_D]
    supp_ref[0] = fo[:, _D:]


def _conv1(nbr, vert_pad, dir0, w1, b1r, nb):
    blkmap = lambda b, n: (b, n, 0)
    return pl.pallas_call(
        _conv1_body,
        grid=(nb, _N // _BLK),
        in_specs=[
            pl.BlockSpec((_TOP, 1, _BLK, _PC), lambda b, n: (0, b, n, 0)),
            pl.BlockSpec((1, _BLK, _PC), blkmap),
            pl.BlockSpec((3, _D), lambda b, n: (0, 0)),
            pl.BlockSpec((_D, 2 * _D), lambda b, n: (0, 0)),
            pl.BlockSpec((1, 2 * _D), lambda b, n: (0, 0)),
        ],
        out_specs=[
            pl.BlockSpec((1, _BLK, _D), blkmap),
            pl.BlockSpec((1, _BLK, _D), blkmap),
            pl.BlockSpec((_NBR, 1, _BLK, _PCS), lambda b, n: (0, b, n, 0)),
        ],
        out_shape=[
            jax.ShapeDtypeStruct((nb, _N, _D), jnp.float32),
            jax.ShapeDtypeStruct((nb, _N, _D), jnp.float32),
            jax.ShapeDtypeStruct((_NBR, nb, _N, _PCS), jnp.float32),
        ],
    )(nbr, vert_pad, dir0, w1, b1r)


def _conv2_body(ndn_ref, sn_ref, cen_ref, dir1_ref, fc1w_ref, fc1b_ref,
                wq_ref, wk_ref, wv_ref, q_ref, kv_ref, fm2_ref):
    d1 = dir1_ref[...]
    n1 = jnp.sqrt(jnp.sum(d1 * d1, axis=0, keepdims=True))
    d1p = jnp.concatenate(
        [d1 / jnp.maximum(n1, 1e-12),
         jnp.zeros((_PCS - 3, _D), jnp.float32)], axis=0)
    en = _NBR * _BLK
    ndnf = ndn_ref[...].reshape(_NBR, _BLK, _PCS).reshape(en, _PCS)
    snf = sn_ref[...].reshape(_NBR, _BLK, _D).reshape(en, _D)
    a = jnp.maximum(_mm(ndnf, d1p), 0.0) * snf
    acc = None
    for j in range(_NBR):
        acc = (a[j * _BLK:(j + 1) * _BLK] if acc is None
               else jnp.maximum(acc, a[j * _BLK:(j + 1) * _BLK]))
    fm2 = jnp.maximum(cen_ref[0] + acc, 0.0)
    x = _mm(fm2, fc1w_ref[...]) + fc1b_ref[...]
    q_ref[0] = _mm(x, wq_ref[...])
    kv_ref[0] = jnp.concatenate(
        [_mm(x, wk_ref[...]), _mm(x, wv_ref[...])], axis=1)
    fm2_ref[0] = fm2


def _conv2(ndn, sn, cen, dir1, fc1_w, fc1br, wq, wk, wv, nb):
    blkmap = lambda b, n: (b, n, 0)
    wmap = lambda b, n: (0, 0)
    od = jax.ShapeDtypeStruct((nb, _N, _D), jnp.float32)
    return pl.pallas_call(
        _conv2_body,
        grid=(nb, _N // _BLK),
        in_specs=[
            pl.BlockSpec((_NBR, 1, _BLK, _PCS), lambda b, n: (0, b, n, 0)),
            pl.BlockSpec((_NBR, 1, _BLK, _D), lambda b, n: (0, b, n, 0)),
            pl.BlockSpec((1, _BLK, _D), blkmap),
            pl.BlockSpec((3, _D), wmap),
            pl.BlockSpec((_D, _D), wmap),
            pl.BlockSpec((1, _D), wmap),
            pl.BlockSpec((_D, _D), wmap),
            pl.BlockSpec((_D, _D), wmap),
            pl.BlockSpec((_D, _D), wmap),
        ],
        out_specs=[pl.BlockSpec((1, _BLK, _D), blkmap),
                   pl.BlockSpec((1, _BLK, 2 * _D), blkmap),
                   pl.BlockSpec((1, _BLK, _D), blkmap)],
        out_shape=[od, jax.ShapeDtypeStruct((nb, _N, 2 * _D), jnp.float32),
                   od],
    )(ndn, sn, cen, dir1, fc1_w, fc1br, wq, wk, wv)


def _attn_body(nbr_ref, vert_ref, q_ref, kv_ref, fm2_ref,
               fcd1_ref, fcd1b_ref, fcd2_ref, fcd2b_ref,
               fcg1_ref, fcg1b_ref, fcg2_ref, fcg2b_ref,
               fc2w_ref, fc2b_ref, out_ref):
    vert = vert_ref[0]
    fd1p = jnp.concatenate(
        [fcd1_ref[...], jnp.zeros((_PC - 3, _D), jnp.float32)], axis=0)
    q = q_ref[0]
    scale = 1.0 / np.sqrt(float(_D))
    ek = _KATT * _BLK
    knnf = nbr_ref[0:_KATT].reshape(_KATT, _BLK, _PC).reshape(ek, _PC)
    vertf = jnp.broadcast_to(vert[None], (_KATT, _BLK, _PC)).reshape(ek, _PC)
    qf = jnp.broadcast_to(q[None], (_KATT, _BLK, _D)).reshape(ek, _D)
    kvf = kv_ref[...].reshape(_KATT, _BLK, 2 * _D).reshape(ek, 2 * _D)
    kkf = kvf[:, :_D]
    vvf = kvf[:, _D:]
    rel = vertf - knnf
    h = jnp.maximum(_mm(rel, fd1p) + fcd1b_ref[...], 0.0)
    pos = _mm(h, fcd2_ref[...]) + fcd2b_ref[...]
    t = qf - kkf + pos
    g = jnp.maximum(_mm(t, fcg1_ref[...]) + fcg1b_ref[...], 0.0)
    z = (_mm(g, fcg2_ref[...]) + fcg2b_ref[...]) * scale
    pv = vvf + pos
    zs = [z[j * _BLK:(j + 1) * _BLK] for j in range(_KATT)]
    m = None
    for j in range(_KATT):
        m = zs[j] if m is None else jnp.maximum(m, zs[j])
    es, s = [], None
    for j in range(_KATT):
        e = jnp.exp(zs[j] - m)
        es.append(e)
        s = e if s is None else s + e
    res = None
    for j in range(_KATT):
        c = (es[j] / s) * pv[j * _BLK:(j + 1) * _BLK]
        res = c if res is None else res + c
    out_ref[0] = (_mm(res, fc2w_ref[...])
                  + fc2b_ref[...] + fm2_ref[0])


def _attn(nbr, vert_pad, q, kv, fm2,
          fcd1, fcd1br, fcd2, fcd2br, fcg1, fcg1br, fcg2, fcg2br,
          fc2_w, fc2br, nb):
    blkmap = lambda b, n: (b, n, 0)
    wmap = lambda b, n: (0, 0)
    return pl.pallas_call(
        _attn_body,
        grid=(nb, _N // _BLK),
        in_specs=[
            pl.BlockSpec((_TOP, 1, _BLK, _PC), lambda b, n: (0, b, n, 0)),
            pl.BlockSpec((1, _BLK, _PC), blkmap),
            pl.BlockSpec((1, _BLK, _D), blkmap),
            pl.BlockSpec((_KATT, 1, _BLK, 2 * _D),
                         lambda b, n: (0, b, n, 0)),
            pl.BlockSpec((1, _BLK, _D), blkmap),
            pl.BlockSpec((3, _D), wmap),
            pl.BlockSpec((1, _D), wmap),
            pl.BlockSpec((_D, _D), wmap),
            pl.BlockSpec((1, _D), wmap),
            pl.BlockSpec((_D, _D), wmap),
            pl.BlockSpec((1, _D), wmap),
            pl.BlockSpec((_D, _D), wmap),
            pl.BlockSpec((1, _D), wmap),
            pl.BlockSpec((_D, _D), wmap),
            pl.BlockSpec((1, _D), wmap),
        ],
        out_specs=pl.BlockSpec((1, _BLK, _D), blkmap),
        out_shape=jax.ShapeDtypeStruct((nb, _N, _D), jnp.float32),
    )(nbr, vert_pad, q, kv, fm2,
      fcd1, fcd1br, fcd2, fcd2br, fcg1, fcg1br, fcg2, fcg2br,
      fc2_w, fc2br)


def kernel(xyz, dir0, w1, b1, dir1, fc1_w, fc1_b, fc2_w, fc2_b,
           fcd1_w, fcd1_b, fcd2_w, fcd2_b, fcg1_w, fcg1_b, fcg2_w, fcg2_b,
           wq, wk, wv):
    vertices = jnp.transpose(xyz, (0, 2, 1))
    vert_pad = jnp.pad(vertices, ((0, 0), (0, 0), (0, _PC - 3)))
    w1b = w1.astype(jnp.bfloat16)
    fc1wb = fc1_w.astype(jnp.bfloat16)
    wqb, wkb, wvb = (wq.astype(jnp.bfloat16), wk.astype(jnp.bfloat16),
                     wv.astype(jnp.bfloat16))
    b1r, fc1br = b1.reshape(1, -1), fc1_b.reshape(1, -1)

    nh = 2
    outs = []
    for h in range(_B // nh):
        sl = slice(h * nh, (h + 1) * nh)
        xyz_h, vert_h, vp_h = xyz[sl], vertices[sl], vert_pad[sl]

        idx = _topk(xyz_h, vert_h, nh)

        idx21 = jnp.transpose(idx[:, :, :_TOP], (2, 0, 1)).reshape(-1)
        (nbr_flat,) = _sc_gather([vp_h.reshape(nh * _N, _PC)], idx21, ns=3)
        nbr = nbr_flat.reshape(_TOP, nh, _N, _PC)

        cen, supp, ndn = _conv1(nbr, vp_h, dir0, w1b, b1r, nh)

        idx20 = jnp.transpose(idx[:, :, 1:_TOP], (2, 0, 1)).reshape(-1)
        (sn_flat,) = _sc_gather([supp.reshape(nh * _N, _D)], idx20, ns=4)
        sn = sn_flat.reshape(_NBR, nh, _N, _D)

        q, kvx, fm2 = _conv2(ndn, sn, cen, dir1, fc1wb, fc1br,
                             wqb, wkb, wvb, nh)

        idx16 = jnp.transpose(idx[:, :, :_KATT], (2, 0, 1)).reshape(-1)
        (kvf,) = _sc_gather([kvx.reshape(nh * _N, 2 * _D)], idx16, ns=2)
        kv = kvf.reshape(_KATT, nh, _N, 2 * _D)

        outs.append(_attn(
            nbr, vp_h, q, kv, fm2,
            fcd1_w, fcd1_b.reshape(1, -1),
            fcd2_w.astype(jnp.bfloat16), fcd2_b.reshape(1, -1),
            fcg1_w.astype(jnp.bfloat16), fcg1_b.reshape(1, -1),
            fcg2_w.astype(jnp.bfloat16), fcg2_b.reshape(1, -1),
            fc2_w.astype(jnp.bfloat16), fc2_b.reshape(1, -1), nh))
    out = jnp.concatenate(outs, axis=0)
    return jnp.transpose(out, (0, 2, 1))
